# Initial kernel scaffold; baseline (speedup 1.0000x reference)
import jax
import jax.numpy as jnp
from jax.experimental import pallas as pl


def kernel(x, edge_attr, edge_index, W_emb, b_emb, W_msg, b_msg, W_upd, b_upd, W_eu, b_eu, W_np, b_np, W_ep, b_ep):
    raise NotImplementedError("write your pallas kernel here")


# speedup vs baseline: 1.6353x; 1.6353x over previous
import functools

import jax
import jax.numpy as jnp
from jax import lax
from jax.experimental import pallas as pl
from jax.experimental.pallas import tpu as pltpu
from jax.experimental.pallas import tpu_sc as plsc

N = 10000
E = 320000
EMB = 128
ED = 16
OUT_DIM = 128

NC = 2
NS = 16
NW = NC * NS
EPW = E // NW
CK = 80
NCHUNK = EPW // CK
RPT = N // NS
ZROWS = 125

_mesh = plsc.VectorSubcoreMesh(
    core_axis_name="c", subcore_axis_name="s", num_cores=NC, num_subcores=NS)

_f32 = jnp.float32


_BM_N = 2000
_BM_E = 2000


def _full(shape):
    return pl.BlockSpec(shape, lambda i: (0,) * len(shape))


def _rows(shape):
    return pl.BlockSpec(shape, lambda i: (i,) + (0,) * (len(shape) - 1))


def _node_emb_proj(x, We, be, A, B, Wm1, bm):
    def body(x_r, we_r, be_r, a_r, b_r, wm_r, bm_r, h_r, p_r, q_r, g_r):
        h = jnp.dot(x_r[...], we_r[...], preferred_element_type=_f32) + be_r[...]
        h_r[...] = h
        p_r[...] = jnp.dot(h, a_r[...], preferred_element_type=_f32)
        q_r[...] = jnp.dot(h, b_r[...], preferred_element_type=_f32)
        g_r[...] = jnp.dot(h, wm_r[...], preferred_element_type=_f32) + bm_r[...]
    return pl.pallas_call(
        body,
        grid=(N // _BM_N,),
        in_specs=[_rows((_BM_N, EMB)), _full((EMB, EMB)), _full((1, EMB)),
                  _full((EMB, ED)), _full((EMB, ED)), _full((EMB, EMB)),
                  _full((1, EMB))],
        out_specs=[_rows((_BM_N, EMB)), _rows((_BM_N, ED)), _rows((_BM_N, ED)),
                   _rows((_BM_N, EMB))],
        out_shape=[jax.ShapeDtypeStruct((N, EMB), _f32),
                   jax.ShapeDtypeStruct((N, ED), _f32),
                   jax.ShapeDtypeStruct((N, ED), _f32),
                   jax.ShapeDtypeStruct((N, EMB), _f32)],
    )(x, We, be, A, B, Wm1, bm)


def _node_update_proj(h, pagg, Wu1, Wu2, bu, A, B, Wm1, bm):
    def body(h_r, pa_r, wu1_r, wu2_r, bu_r, a_r, b_r, wm_r, bm_r,
             hn_r, p_r, q_r, g_r):
        agg = pa_r[0] + pa_r[1]
        hn = jnp.maximum(
            jnp.dot(h_r[...], wu1_r[...], preferred_element_type=_f32)
            + jnp.dot(agg, wu2_r[...], preferred_element_type=_f32)
            + bu_r[...], 0.0)
        hn_r[...] = hn
        p_r[...] = jnp.dot(hn, a_r[...], preferred_element_type=_f32)
        q_r[...] = jnp.dot(hn, b_r[...], preferred_element_type=_f32)
        g_r[...] = jnp.dot(hn, wm_r[...], preferred_element_type=_f32) + bm_r[...]
    return pl.pallas_call(
        body,
        grid=(N // _BM_N,),
        in_specs=[_rows((_BM_N, EMB)),
                  pl.BlockSpec((NC, _BM_N, EMB), lambda i: (0, i, 0)),
                  _full((EMB, EMB)), _full((EMB, EMB)), _full((1, EMB)),
                  _full((EMB, ED)), _full((EMB, ED)), _full((EMB, EMB)),
                  _full((1, EMB))],
        out_specs=[_rows((_BM_N, EMB)), _rows((_BM_N, ED)), _rows((_BM_N, ED)),
                   _rows((_BM_N, EMB))],
        out_shape=[jax.ShapeDtypeStruct((N, EMB), _f32),
                   jax.ShapeDtypeStruct((N, ED), _f32),
                   jax.ShapeDtypeStruct((N, ED), _f32),
                   jax.ShapeDtypeStruct((N, EMB), _f32)],
    )(h, pagg, Wu1, Wu2, bu, A, B, Wm1, bm)


def _node_final(h, pagg, Wu1, Wu2, bu, Wnp, bnp, WS, WD, bD):
    def body(h_r, pa_r, wu1_r, wu2_r, bu_r, wnp_r, bnp_r, ws_r, wd_r, bd_r,
             np_r, s_r, d_r):
        agg = pa_r[0] + pa_r[1]
        hn = jnp.maximum(
            jnp.dot(h_r[...], wu1_r[...], preferred_element_type=_f32)
            + jnp.dot(agg, wu2_r[...], preferred_element_type=_f32)
            + bu_r[...], 0.0)
        np_r[...] = jnp.dot(hn, wnp_r[...], preferred_element_type=_f32) + bnp_r[...]
        s_r[...] = jnp.dot(hn, ws_r[...], preferred_element_type=_f32)
        d_r[...] = jnp.dot(hn, wd_r[...], preferred_element_type=_f32) + bd_r[...]
    return pl.pallas_call(
        body,
        grid=(N // _BM_N,),
        in_specs=[_rows((_BM_N, EMB)),
                  pl.BlockSpec((NC, _BM_N, EMB), lambda i: (0, i, 0)),
                  _full((EMB, EMB)), _full((EMB, EMB)), _full((1, EMB)),
                  _full((EMB, OUT_DIM)), _full((1, OUT_DIM)),
                  _full((EMB, 2 * ED)), _full((EMB, 2 * ED)), _full((1, 2 * ED))],
        out_specs=[_rows((_BM_N, OUT_DIM)), _rows((_BM_N, 2 * ED)),
                   _rows((_BM_N, 2 * ED))],
        out_shape=[jax.ShapeDtypeStruct((N, OUT_DIM), _f32),
                   jax.ShapeDtypeStruct((N, 2 * ED), _f32),
                   jax.ShapeDtypeStruct((N, 2 * ED), _f32)],
    )(h, pagg, Wu1, Wu2, bu, Wnp, bnp, WS, WD, bD)


def _edge_proj_first(ea, C, beu):
    def body(ea_r, c_r, b_r, out_r):
        out_r[...] = jnp.dot(ea_r[...], c_r[...],
                             preferred_element_type=_f32) + b_r[...]
    return pl.pallas_call(
        body,
        grid=(E // _BM_E,),
        in_specs=[_rows((_BM_E, ED)), _full((ED, ED)), _full((1, ED))],
        out_specs=_rows((_BM_E, ED)),
        out_shape=jax.ShapeDtypeStruct((E, ED), _f32),
    )(ea, C, beu)


def _edge_proj(ea, Wm2, C, beu):
    def body(ea_r, wm_r, c_r, b_r, t_r, cc_r):
        v = ea_r[...]
        t_r[...] = jnp.dot(v, wm_r[...], preferred_element_type=_f32)
        cc_r[...] = jnp.dot(v, c_r[...], preferred_element_type=_f32) + b_r[...]
    return pl.pallas_call(
        body,
        grid=(E // _BM_E,),
        in_specs=[_rows((_BM_E, ED)), _full((ED, EMB)), _full((ED, ED)),
                  _full((1, ED))],
        out_specs=[_rows((_BM_E, EMB)), _rows((_BM_E, ED))],
        out_shape=[jax.ShapeDtypeStruct((E, EMB), _f32),
                   jax.ShapeDtypeStruct((E, ED), _f32)],
    )(ea, Wm2, C, beu)


def _edge_head_sum(z):
    def body(z_r, out_r):
        out_r[...] = jnp.sum(z_r[...], axis=1, keepdims=True)
    return pl.pallas_call(
        body,
        grid=(E // _BM_E,),
        in_specs=[_rows((_BM_E, ED))],
        out_specs=_rows((_BM_E, 1)),
        out_shape=jax.ShapeDtypeStruct((E, 1), _f32),
    )(z)


@functools.partial(
    pl.kernel,
    out_type=jax.ShapeDtypeStruct((E, ED), _f32),
    mesh=_mesh,
    compiler_params=pltpu.CompilerParams(use_tc_tiling_on_sc=False),
    scratch_types=[
        pltpu.VMEM((CK,), jnp.int32),
        pltpu.VMEM((CK,), jnp.int32),
        pltpu.VMEM((CK, ED), _f32),
        pltpu.VMEM((CK, ED), _f32),
        pltpu.VMEM((CK, ED), _f32),
        pltpu.SemaphoreType.DMA,
    ],
)
def _sc_edge_update(p_hbm, q_hbm, c_hbm, src_hbm, dst_hbm, out_hbm,
                    idx_s, idx_d, ps, qd, cb, sem):
    wid = lax.axis_index("c") * NS + lax.axis_index("s")
    base = wid * EPW

    def chunk(i, carry):
        off = pl.multiple_of(base + i * CK, 8)
        pltpu.sync_copy(src_hbm.at[pl.ds(off, CK)], idx_s)
        pltpu.sync_copy(dst_hbm.at[pl.ds(off, CK)], idx_d)
        pltpu.async_copy(p_hbm.at[idx_s], ps, sem).wait()
        pltpu.async_copy(q_hbm.at[idx_d], qd, sem).wait()
        pltpu.sync_copy(c_hbm.at[pl.ds(off, CK)], cb)

        def ebody(k, c2):
            cb[k] = jnp.maximum(ps[k] + qd[k] + cb[k], 0.0)
            return c2
        lax.fori_loop(0, CK, ebody, 0)
        pltpu.sync_copy(cb, out_hbm.at[pl.ds(off, CK)])
        return carry

    lax.fori_loop(0, NCHUNK, chunk, 0)


@functools.partial(
    pl.kernel,
    out_type=jax.ShapeDtypeStruct((E, 16), _f32),
    mesh=_mesh,
    compiler_params=pltpu.CompilerParams(use_tc_tiling_on_sc=False),
    scratch_types=[
        pltpu.VMEM((CK,), jnp.int32),
        pltpu.VMEM((CK,), jnp.int32),
        pltpu.VMEM((CK, 2 * ED), _f32),
        pltpu.VMEM((CK, 2 * ED), _f32),
        pltpu.VMEM((CK, ED), _f32),
        pltpu.VMEM((CK, 16), _f32),
        pltpu.VMEM((16,), _f32),
        pltpu.SemaphoreType.DMA,
    ],
)
def _sc_edge_pred(s_hbm, d_hbm, c_hbm, u3_hbm, src_hbm, dst_hbm, out_hbm,
                  idx_s, idx_d, ps, qd, cb, ob, u3v, sem):
    wid = lax.axis_index("c") * NS + lax.axis_index("s")
    base = wid * EPW
    pltpu.sync_copy(u3_hbm, u3v)

    def chunk(i, carry):
        off = pl.multiple_of(base + i * CK, 8)
        pltpu.sync_copy(src_hbm.at[pl.ds(off, CK)], idx_s)
        pltpu.sync_copy(dst_hbm.at[pl.ds(off, CK)], idx_d)
        pltpu.async_copy(s_hbm.at[idx_s], ps, sem).wait()
        pltpu.async_copy(d_hbm.at[idx_d], qd, sem).wait()
        pltpu.sync_copy(c_hbm.at[pl.ds(off, CK)], cb)

        def ebody(k, c2):
            pa = ps[k, pl.ds(0, ED)]
            pb = ps[k, pl.ds(ED, ED)]
            qa = qd[k, pl.ds(0, ED)]
            qb = qd[k, pl.ds(ED, ED)]
            ea2 = jnp.maximum(pa + qa + cb[k], 0.0)
            ob[k] = ea2 * u3v[...] + pb + qb
            return c2
        lax.fori_loop(0, CK, ebody, 0)
        pltpu.sync_copy(ob, out_hbm.at[pl.ds(off, CK)])
        return carry

    lax.fori_loop(0, NCHUNK, chunk, 0)


@functools.partial(
    pl.kernel,
    out_type=jax.ShapeDtypeStruct((NC, N, EMB), _f32),
    mesh=_mesh,
    compiler_params=pltpu.CompilerParams(use_tc_tiling_on_sc=False),
    scratch_types=[
        pltpu.VMEM_SHARED((N, EMB), _f32),
        pltpu.VMEM((ZROWS, EMB), _f32),
        pltpu.VMEM((CK,), jnp.int32),
        pltpu.VMEM((CK,), jnp.int32),
        pltpu.VMEM((CK, EMB), _f32),
        pltpu.VMEM((CK, EMB), _f32),
        pltpu.SemaphoreType.DMA,
    ],
)
def _sc_msg_agg(g_hbm, t_hbm, src_hbm, dst_hbm, out_hbm,
                agg_sh, zb, idx_s, idx_d, rows, tb, sem):
    cid = lax.axis_index("c")
    sid = lax.axis_index("s")
    wid = cid * NS + sid
    base = wid * EPW

    def zrow(k, carry):
        for j in range(EMB // 16):
            zb[k, pl.ds(j * 16, 16)] = jnp.zeros((16,), _f32)
        return carry
    lax.fori_loop(0, ZROWS, zrow, 0)

    def zcp(i, carry):
        pltpu.sync_copy(zb, agg_sh.at[pl.ds(sid * RPT + i * ZROWS, ZROWS)])
        return carry
    lax.fori_loop(0, RPT // ZROWS, zcp, 0)
    plsc.subcore_barrier()

    def chunk(i, carry):
        off = pl.multiple_of(base + i * CK, 8)
        pltpu.sync_copy(dst_hbm.at[pl.ds(off, CK)], idx_d)
        pltpu.sync_copy(src_hbm.at[pl.ds(off, CK)], idx_s)
        pltpu.async_copy(g_hbm.at[idx_d], rows, sem).wait()
        pltpu.sync_copy(t_hbm.at[pl.ds(off, CK)], tb)

        def ebody(k, c2):
            for j in range(EMB // 16):
                sl = pl.ds(j * 16, 16)
                rows[k, sl] = jnp.maximum(rows[k, sl] + tb[k, sl], 0.0)
            return c2
        lax.fori_loop(0, CK, ebody, 0)
        pltpu.sync_copy(rows, agg_sh.at[idx_s], add=True)
        return carry

    lax.fori_loop(0, NCHUNK, chunk, 0)
    plsc.subcore_barrier()
    pltpu.sync_copy(agg_sh.at[pl.ds(sid * RPT, RPT)],
                    out_hbm.at[cid, pl.ds(sid * RPT, RPT)])


def kernel(x, edge_attr, edge_index, W_emb, b_emb, W_msg, b_msg, W_upd, b_upd,
           W_eu, b_eu, W_np, b_np, W_ep, b_ep):
    src = edge_index[0]
    dst = edge_index[1]

    def r2(b):
        return b.reshape(1, -1)

    A = [W_eu[l][:EMB] for l in range(3)]
    B = [W_eu[l][EMB:2 * EMB] for l in range(3)]
    C = [W_eu[l][2 * EMB:] for l in range(3)]
    Wm1 = [W_msg[l][:EMB] for l in range(2)]
    Wm2 = [W_msg[l][EMB:] for l in range(2)]
    Wu1 = [W_upd[l][:EMB] for l in range(2)]
    Wu2 = [W_upd[l][EMB:] for l in range(2)]

    zpad = jnp.zeros((EMB, ED - 1), _f32)
    WS = jnp.concatenate([A[2], W_ep[:EMB], zpad], axis=1)
    WD = jnp.concatenate([B[2], W_ep[EMB:2 * EMB], zpad], axis=1)
    bD = jnp.zeros((2 * ED,), _f32).at[ED + 1].set(b_ep[0])
    u3 = W_ep[2 * EMB:, 0]

    h0, P0, Q0, g0 = _node_emb_proj(x, W_emb, r2(b_emb), A[0], B[0],
                                    Wm1[0], r2(b_msg[0]))
    c0 = _edge_proj_first(edge_attr, C[0], r2(b_eu[0]))
    ea0 = _sc_edge_update(P0, Q0, c0, src, dst)
    t0, c1 = _edge_proj(ea0, Wm2[0], C[1], r2(b_eu[1]))
    pagg0 = _sc_msg_agg(g0, t0, src, dst)

    h1, P1, Q1, g1 = _node_update_proj(h0, pagg0, Wu1[0], Wu2[0],
                                       r2(b_upd[0]), A[1], B[1],
                                       Wm1[1], r2(b_msg[1]))
    ea1 = _sc_edge_update(P1, Q1, c1, src, dst)
    t1, c2 = _edge_proj(ea1, Wm2[1], C[2], r2(b_eu[2]))
    pagg1 = _sc_msg_agg(g1, t1, src, dst)

    npred, S2, D2 = _node_final(h1, pagg1, Wu1[1], Wu2[1], r2(b_upd[1]),
                                W_np, r2(b_np), WS, WD, r2(bD))
    z16 = _sc_edge_pred(S2, D2, c2, u3, src, dst)
    return (npred, _edge_head_sum(z16))


# speedup vs baseline: 2.9664x; 1.8140x over previous
import functools

import jax
import jax.numpy as jnp
from jax import lax
from jax.experimental import pallas as pl
from jax.experimental.pallas import tpu as pltpu
from jax.experimental.pallas import tpu_sc as plsc

N = 10000
E = 320000
EMB = 128
ED = 16
OUT_DIM = 128

NC = 2
NS = 16
NW = NC * NS
EPW = E // NW
CK = 80
NCHUNK = EPW // CK
NBUF = 4
CKM = 40
NCHM = EPW // CKM
NBUFM = 2
RPT = N // NS
ZROWS = 25

_mesh = plsc.VectorSubcoreMesh(
    core_axis_name="c", subcore_axis_name="s", num_cores=NC, num_subcores=NS)

_f32 = jnp.float32


_BM_N = 2000
_BM_E = 2000


def _full(shape):
    return pl.BlockSpec(shape, lambda i: (0,) * len(shape))


def _rows(shape):
    return pl.BlockSpec(shape, lambda i: (i,) + (0,) * (len(shape) - 1))


def _node_emb_proj(x, We, be, A, B, Wm1, bm):
    def body(x_r, we_r, be_r, a_r, b_r, wm_r, bm_r, h_r, p_r, q_r, g_r):
        h = jnp.dot(x_r[...], we_r[...], preferred_element_type=_f32) + be_r[...]
        h_r[...] = h
        p_r[...] = jnp.dot(h, a_r[...], preferred_element_type=_f32)
        q_r[...] = jnp.dot(h, b_r[...], preferred_element_type=_f32)
        g_r[...] = jnp.dot(h, wm_r[...], preferred_element_type=_f32) + bm_r[...]
    return pl.pallas_call(
        body,
        grid=(N // _BM_N,),
        in_specs=[_rows((_BM_N, EMB)), _full((EMB, EMB)), _full((1, EMB)),
                  _full((EMB, ED)), _full((EMB, ED)), _full((EMB, EMB)),
                  _full((1, EMB))],
        out_specs=[_rows((_BM_N, EMB)), _rows((_BM_N, ED)), _rows((_BM_N, ED)),
                   _rows((_BM_N, EMB))],
        out_shape=[jax.ShapeDtypeStruct((N, EMB), _f32),
                   jax.ShapeDtypeStruct((N, ED), _f32),
                   jax.ShapeDtypeStruct((N, ED), _f32),
                   jax.ShapeDtypeStruct((N, EMB), _f32)],
    )(x, We, be, A, B, Wm1, bm)


def _node_update_proj(h, pagg, Wu1, Wu2, bu, A, B, Wm1, bm):
    def body(h_r, pa_r, wu1_r, wu2_r, bu_r, a_r, b_r, wm_r, bm_r,
             hn_r, p_r, q_r, g_r):
        agg = pa_r[0] + pa_r[1]
        hn = jnp.maximum(
            jnp.dot(h_r[...], wu1_r[...], preferred_element_type=_f32)
            + jnp.dot(agg, wu2_r[...], preferred_element_type=_f32)
            + bu_r[...], 0.0)
        hn_r[...] = hn
        p_r[...] = jnp.dot(hn, a_r[...], preferred_element_type=_f32)
        q_r[...] = jnp.dot(hn, b_r[...], preferred_element_type=_f32)
        g_r[...] = jnp.dot(hn, wm_r[...], preferred_element_type=_f32) + bm_r[...]
    return pl.pallas_call(
        body,
        grid=(N // _BM_N,),
        in_specs=[_rows((_BM_N, EMB)),
                  pl.BlockSpec((NC, _BM_N, EMB), lambda i: (0, i, 0)),
                  _full((EMB, EMB)), _full((EMB, EMB)), _full((1, EMB)),
                  _full((EMB, ED)), _full((EMB, ED)), _full((EMB, EMB)),
                  _full((1, EMB))],
        out_specs=[_rows((_BM_N, EMB)), _rows((_BM_N, ED)), _rows((_BM_N, ED)),
                   _rows((_BM_N, EMB))],
        out_shape=[jax.ShapeDtypeStruct((N, EMB), _f32),
                   jax.ShapeDtypeStruct((N, ED), _f32),
                   jax.ShapeDtypeStruct((N, ED), _f32),
                   jax.ShapeDtypeStruct((N, EMB), _f32)],
    )(h, pagg, Wu1, Wu2, bu, A, B, Wm1, bm)


def _node_final(h, pagg, Wu1, Wu2, bu, Wnp, bnp, WS, WD, bD):
    def body(h_r, pa_r, wu1_r, wu2_r, bu_r, wnp_r, bnp_r, ws_r, wd_r, bd_r,
             np_r, s_r, d_r):
        agg = pa_r[0] + pa_r[1]
        hn = jnp.maximum(
            jnp.dot(h_r[...], wu1_r[...], preferred_element_type=_f32)
            + jnp.dot(agg, wu2_r[...], preferred_element_type=_f32)
            + bu_r[...], 0.0)
        np_r[...] = jnp.dot(hn, wnp_r[...], preferred_element_type=_f32) + bnp_r[...]
        s_r[...] = jnp.dot(hn, ws_r[...], preferred_element_type=_f32)
        d_r[...] = jnp.dot(hn, wd_r[...], preferred_element_type=_f32) + bd_r[...]
    return pl.pallas_call(
        body,
        grid=(N // _BM_N,),
        in_specs=[_rows((_BM_N, EMB)),
                  pl.BlockSpec((NC, _BM_N, EMB), lambda i: (0, i, 0)),
                  _full((EMB, EMB)), _full((EMB, EMB)), _full((1, EMB)),
                  _full((EMB, OUT_DIM)), _full((1, OUT_DIM)),
                  _full((EMB, 2 * ED)), _full((EMB, 2 * ED)), _full((1, 2 * ED))],
        out_specs=[_rows((_BM_N, OUT_DIM)), _rows((_BM_N, 2 * ED)),
                   _rows((_BM_N, 2 * ED))],
        out_shape=[jax.ShapeDtypeStruct((N, OUT_DIM), _f32),
                   jax.ShapeDtypeStruct((N, 2 * ED), _f32),
                   jax.ShapeDtypeStruct((N, 2 * ED), _f32)],
    )(h, pagg, Wu1, Wu2, bu, Wnp, bnp, WS, WD, bD)


def _edge_proj_first(ea, C, beu):
    def body(ea_r, c_r, b_r, out_r):
        out_r[...] = jnp.dot(ea_r[...], c_r[...],
                             preferred_element_type=_f32) + b_r[...]
    return pl.pallas_call(
        body,
        grid=(E // _BM_E,),
        in_specs=[_rows((_BM_E, ED)), _full((ED, ED)), _full((1, ED))],
        out_specs=_rows((_BM_E, ED)),
        out_shape=jax.ShapeDtypeStruct((E, ED), _f32),
    )(ea, C, beu)


def _edge_proj(ea, Wm2, C, beu):
    def body(ea_r, wm_r, c_r, b_r, t_r, cc_r):
        v = ea_r[...]
        t_r[...] = jnp.dot(v, wm_r[...], preferred_element_type=_f32)
        cc_r[...] = jnp.dot(v, c_r[...], preferred_element_type=_f32) + b_r[...]
    return pl.pallas_call(
        body,
        grid=(E // _BM_E,),
        in_specs=[_rows((_BM_E, ED)), _full((ED, EMB)), _full((ED, ED)),
                  _full((1, ED))],
        out_specs=[_rows((_BM_E, EMB)), _rows((_BM_E, ED))],
        out_shape=[jax.ShapeDtypeStruct((E, EMB), _f32),
                   jax.ShapeDtypeStruct((E, ED), _f32)],
    )(ea, Wm2, C, beu)


def _edge_head_sum(z):
    def body(z_r, out_r):
        out_r[...] = jnp.sum(z_r[...], axis=1, keepdims=True)
    return pl.pallas_call(
        body,
        grid=(E // _BM_E,),
        in_specs=[_rows((_BM_E, ED))],
        out_specs=_rows((_BM_E, 1)),
        out_shape=jax.ShapeDtypeStruct((E, 1), _f32),
    )(z)


def _sc_params():
    return pltpu.CompilerParams(use_tc_tiling_on_sc=False)


def _chunk_off(base, ci):
    return pl.multiple_of(base + ci * CK, 8)


@functools.partial(
    pl.kernel,
    out_type=jax.ShapeDtypeStruct((E, ED), _f32),
    mesh=_mesh,
    compiler_params=_sc_params(),
    scratch_types=(
        [pltpu.VMEM((NCHUNK, CK), jnp.int32)] * 2
        + [pltpu.VMEM((CK, ED), _f32)] * (4 * NBUF)
        + [pltpu.SemaphoreType.DMA] * (4 * NBUF)
    ),
)
def _sc_edge_update(p_hbm, q_hbm, c_hbm, src3_hbm, dst3_hbm, out_hbm, *sc):
    srcb, dstb = sc[0], sc[1]
    PS = sc[2:2 + NBUF]
    QD = sc[2 + NBUF:2 + 2 * NBUF]
    CB = sc[2 + 2 * NBUF:2 + 3 * NBUF]
    OB = sc[2 + 3 * NBUF:2 + 4 * NBUF]
    base_s = 2 + 4 * NBUF
    SP = sc[base_s:base_s + NBUF]
    SQ = sc[base_s + NBUF:base_s + 2 * NBUF]
    SCM = sc[base_s + 2 * NBUF:base_s + 3 * NBUF]
    SW = sc[base_s + 3 * NBUF:base_s + 4 * NBUF]

    wid = lax.axis_index("c") * NS + lax.axis_index("s")
    base = wid * EPW
    pltpu.sync_copy(src3_hbm.at[wid], srcb)
    pltpu.sync_copy(dst3_hbm.at[wid], dstb)

    def issue(ci, b):
        off = _chunk_off(base, ci)
        pltpu.async_copy(p_hbm.at[srcb.at[ci]], PS[b], SP[b])
        pltpu.async_copy(q_hbm.at[dstb.at[ci]], QD[b], SQ[b])
        pltpu.async_copy(c_hbm.at[pl.ds(off, CK)], CB[b], SCM[b])

    def wait_wb(b, off):
        pltpu.make_async_copy(OB[b], out_hbm.at[pl.ds(off, CK)], SW[b]).wait()

    for b in range(NBUF):
        issue(b, b)

    def step(ci, b, wb_cond, issue_next):
        off = _chunk_off(base, ci)
        pltpu.make_async_copy(p_hbm.at[srcb.at[ci]], PS[b], SP[b]).wait()
        pltpu.make_async_copy(q_hbm.at[dstb.at[ci]], QD[b], SQ[b]).wait()
        pltpu.make_async_copy(c_hbm.at[pl.ds(off, CK)], CB[b], SCM[b]).wait()
        if wb_cond is True:
            wait_wb(b, off)
        elif wb_cond is not None:
            pl.when(wb_cond)(lambda: wait_wb(b, off))

        def ebody(k, c2):
            OB[b][k] = jnp.maximum(PS[b][k] + QD[b][k] + CB[b][k], 0.0)
            return c2
        lax.fori_loop(0, CK, ebody, 0)
        pltpu.async_copy(OB[b], out_hbm.at[pl.ds(off, CK)], SW[b])
        if issue_next:
            pl.when(ci + NBUF < NCHUNK)(lambda: issue(ci + NBUF, b))

    def outer(io, c2):
        for b in range(NBUF):
            step(NBUF * io + b, b, wb_cond=(io >= 1), issue_next=True)
        return c2

    lax.fori_loop(0, (NCHUNK - 1) // NBUF, outer, 0)
    step(NCHUNK - 1, (NCHUNK - 1) % NBUF, wb_cond=True, issue_next=False)
    for b in range(NBUF):
        wait_wb(b, base)


@functools.partial(
    pl.kernel,
    out_type=jax.ShapeDtypeStruct((E, 16), _f32),
    mesh=_mesh,
    compiler_params=_sc_params(),
    scratch_types=(
        [pltpu.VMEM((NCHUNK, CK), jnp.int32)] * 2
        + [pltpu.VMEM((CK, 2 * ED), _f32)] * (2 * NBUF)
        + [pltpu.VMEM((CK, ED), _f32)] * (2 * NBUF)
        + [pltpu.VMEM((16,), _f32)]
        + [pltpu.SemaphoreType.DMA] * (4 * NBUF)
    ),
)
def _sc_edge_pred(s_hbm, d_hbm, c_hbm, u3_hbm, src3_hbm, dst3_hbm, out_hbm,
                  *sc):
    srcb, dstb = sc[0], sc[1]
    PS = sc[2:2 + NBUF]
    QD = sc[2 + NBUF:2 + 2 * NBUF]
    CB = sc[2 + 2 * NBUF:2 + 3 * NBUF]
    OB = sc[2 + 3 * NBUF:2 + 4 * NBUF]
    u3v = sc[2 + 4 * NBUF]
    base_s = 3 + 4 * NBUF
    SP = sc[base_s:base_s + NBUF]
    SQ = sc[base_s + NBUF:base_s + 2 * NBUF]
    SCM = sc[base_s + 2 * NBUF:base_s + 3 * NBUF]
    SW = sc[base_s + 3 * NBUF:base_s + 4 * NBUF]

    wid = lax.axis_index("c") * NS + lax.axis_index("s")
    base = wid * EPW
    pltpu.sync_copy(src3_hbm.at[wid], srcb)
    pltpu.sync_copy(dst3_hbm.at[wid], dstb)
    pltpu.sync_copy(u3_hbm, u3v)

    def issue(ci, b):
        off = _chunk_off(base, ci)
        pltpu.async_copy(s_hbm.at[srcb.at[ci]], PS[b], SP[b])
        pltpu.async_copy(d_hbm.at[dstb.at[ci]], QD[b], SQ[b])
        pltpu.async_copy(c_hbm.at[pl.ds(off, CK)], CB[b], SCM[b])

    def wait_wb(b, off):
        pltpu.make_async_copy(OB[b], out_hbm.at[pl.ds(off, CK)], SW[b]).wait()

    for b in range(NBUF):
        issue(b, b)

    def step(ci, b, wb_cond, issue_next):
        off = _chunk_off(base, ci)
        pltpu.make_async_copy(s_hbm.at[srcb.at[ci]], PS[b], SP[b]).wait()
        pltpu.make_async_copy(d_hbm.at[dstb.at[ci]], QD[b], SQ[b]).wait()
        pltpu.make_async_copy(c_hbm.at[pl.ds(off, CK)], CB[b], SCM[b]).wait()
        if wb_cond is True:
            wait_wb(b, off)
        elif wb_cond is not None:
            pl.when(wb_cond)(lambda: wait_wb(b, off))

        def ebody(k, c2):
            pa = PS[b][k, pl.ds(0, ED)]
            pb = PS[b][k, pl.ds(ED, ED)]
            qa = QD[b][k, pl.ds(0, ED)]
            qb = QD[b][k, pl.ds(ED, ED)]
            ea2 = jnp.maximum(pa + qa + CB[b][k], 0.0)
            OB[b][k] = ea2 * u3v[...] + pb + qb
            return c2
        lax.fori_loop(0, CK, ebody, 0)
        pltpu.async_copy(OB[b], out_hbm.at[pl.ds(off, CK)], SW[b])
        if issue_next:
            pl.when(ci + NBUF < NCHUNK)(lambda: issue(ci + NBUF, b))

    def outer(io, c2):
        for b in range(NBUF):
            step(NBUF * io + b, b, wb_cond=(io >= 1), issue_next=True)
        return c2

    lax.fori_loop(0, (NCHUNK - 1) // NBUF, outer, 0)
    step(NCHUNK - 1, (NCHUNK - 1) % NBUF, wb_cond=True, issue_next=False)
    for b in range(NBUF):
        wait_wb(b, base)


@functools.partial(
    pl.kernel,
    out_type=jax.ShapeDtypeStruct((NC, N, EMB), _f32),
    mesh=_mesh,
    compiler_params=_sc_params(),
    scratch_types=(
        [pltpu.VMEM_SHARED((N, EMB), _f32)]
        + [pltpu.VMEM((ZROWS, EMB), _f32)]
        + [pltpu.VMEM((NCHM, CKM), jnp.int32)] * 2
        + [pltpu.VMEM((CKM, EMB), _f32)] * (2 * NBUFM)
        + [pltpu.SemaphoreType.DMA] * (2 * NBUFM)
    ),
)
def _sc_msg_agg(g_hbm, t_hbm, src3_hbm, dst3_hbm, out_hbm, *sc):
    agg_sh, zb = sc[0], sc[1]
    srcb, dstb = sc[2], sc[3]
    ROWS = sc[4:4 + NBUFM]
    TB = sc[4 + NBUFM:4 + 2 * NBUFM]
    SG = sc[4 + 2 * NBUFM:4 + 3 * NBUFM]
    ST = sc[4 + 3 * NBUFM:4 + 4 * NBUFM]

    cid = lax.axis_index("c")
    sid = lax.axis_index("s")
    wid = cid * NS + sid
    base = wid * EPW
    pltpu.sync_copy(src3_hbm.at[wid], srcb)
    pltpu.sync_copy(dst3_hbm.at[wid], dstb)

    def zrow(k, carry):
        for j in range(EMB // 16):
            zb[k, pl.ds(j * 16, 16)] = jnp.zeros((16,), _f32)
        return carry
    lax.fori_loop(0, ZROWS, zrow, 0)

    def zcp(i, carry):
        pltpu.sync_copy(zb, agg_sh.at[pl.ds(sid * RPT + i * ZROWS, ZROWS)])
        return carry
    lax.fori_loop(0, RPT // ZROWS, zcp, 0)
    plsc.subcore_barrier()

    def issue(ci, b):
        off = base + ci * CKM
        pltpu.async_copy(g_hbm.at[dstb.at[ci]], ROWS[b], SG[b])
        pltpu.async_copy(t_hbm.at[pl.ds(pl.multiple_of(off, 8), CKM)],
                         TB[b], ST[b])

    for b in range(NBUFM):
        issue(b, b)

    def step(ci, b):
        off = base + ci * CKM
        pltpu.make_async_copy(g_hbm.at[dstb.at[ci]], ROWS[b], SG[b]).wait()
        pltpu.make_async_copy(t_hbm.at[pl.ds(pl.multiple_of(off, 8), CKM)],
                              TB[b], ST[b]).wait()

        def ebody(k, c2):
            for j in range(EMB // 16):
                sl = pl.ds(j * 16, 16)
                ROWS[b][k, sl] = jnp.maximum(ROWS[b][k, sl] + TB[b][k, sl], 0.0)
            return c2
        lax.fori_loop(0, CKM, ebody, 0)
        pltpu.sync_copy(ROWS[b], agg_sh.at[srcb.at[ci]], add=True)
        pl.when(ci + NBUFM < NCHM)(lambda: issue(ci + NBUFM, b))

    def outer(io, c2):
        for b in range(NBUFM):
            step(NBUFM * io + b, b)
        return c2

    lax.fori_loop(0, NCHM // NBUFM, outer, 0)
    plsc.subcore_barrier()
    pltpu.sync_copy(agg_sh.at[pl.ds(sid * RPT, RPT)],
                    out_hbm.at[cid, pl.ds(sid * RPT, RPT)])


def kernel(x, edge_attr, edge_index, W_emb, b_emb, W_msg, b_msg, W_upd, b_upd,
           W_eu, b_eu, W_np, b_np, W_ep, b_ep):
    src = edge_index[0]
    dst = edge_index[1]
    src3 = src.reshape(NW, NCHUNK, CK)
    dst3 = dst.reshape(NW, NCHUNK, CK)
    srcm = src.reshape(NW, NCHM, CKM)
    dstm = dst.reshape(NW, NCHM, CKM)

    def r2(b):
        return b.reshape(1, -1)

    A = [W_eu[l][:EMB] for l in range(3)]
    B = [W_eu[l][EMB:2 * EMB] for l in range(3)]
    C = [W_eu[l][2 * EMB:] for l in range(3)]
    Wm1 = [W_msg[l][:EMB] for l in range(2)]
    Wm2 = [W_msg[l][EMB:] for l in range(2)]
    Wu1 = [W_upd[l][:EMB] for l in range(2)]
    Wu2 = [W_upd[l][EMB:] for l in range(2)]

    zpad = jnp.zeros((EMB, ED - 1), _f32)
    WS = jnp.concatenate([A[2], W_ep[:EMB], zpad], axis=1)
    WD = jnp.concatenate([B[2], W_ep[EMB:2 * EMB], zpad], axis=1)
    bD = jnp.zeros((2 * ED,), _f32).at[ED + 1].set(b_ep[0])
    u3 = W_ep[2 * EMB:, 0]

    h0, P0, Q0, g0 = _node_emb_proj(x, W_emb, r2(b_emb), A[0], B[0],
                                    Wm1[0], r2(b_msg[0]))
    c0 = _edge_proj_first(edge_attr, C[0], r2(b_eu[0]))
    ea0 = _sc_edge_update(P0, Q0, c0, src3, dst3)
    t0, c1 = _edge_proj(ea0, Wm2[0], C[1], r2(b_eu[1]))
    pagg0 = _sc_msg_agg(g0, t0, srcm, dstm)

    h1, P1, Q1, g1 = _node_update_proj(h0, pagg0, Wu1[0], Wu2[0],
                                       r2(b_upd[0]), A[1], B[1],
                                       Wm1[1], r2(b_msg[1]))
    ea1 = _sc_edge_update(P1, Q1, c1, src3, dst3)
    t1, c2 = _edge_proj(ea1, Wm2[1], C[2], r2(b_eu[2]))
    pagg1 = _sc_msg_agg(g1, t1, srcm, dstm)

    npred, S2, D2 = _node_final(h1, pagg1, Wu1[1], Wu2[1], r2(b_upd[1]),
                                W_np, r2(b_np), WS, WD, r2(bD))
    z16 = _sc_edge_pred(S2, D2, c2, u3, src3, dst3)
    return (npred, _edge_head_sum(z16))


# speedup vs baseline: 2.9693x; 1.0010x over previous
import functools

import jax
import jax.numpy as jnp
from jax import lax
from jax.experimental import pallas as pl
from jax.experimental.pallas import tpu as pltpu
from jax.experimental.pallas import tpu_sc as plsc

N = 10000
E = 320000
EMB = 128
ED = 16
OUT_DIM = 128

NC = 2
NS = 16
NW = NC * NS
EPW = E // NW
CK = 80
NCHUNK = EPW // CK
NBUF = 4
CKM = 40
NCHM = EPW // CKM
NBUFM = 2
RPT = N // NS
ZROWS = 25

_mesh = plsc.VectorSubcoreMesh(
    core_axis_name="c", subcore_axis_name="s", num_cores=NC, num_subcores=NS)

_f32 = jnp.float32


_BM_N = 2000
_BM_E = 2000


def _full(shape):
    return pl.BlockSpec(shape, lambda i: (0,) * len(shape))


def _rows(shape):
    return pl.BlockSpec(shape, lambda i: (i,) + (0,) * (len(shape) - 1))


def _node_emb_proj(x, We, be, A, B, Wm1, bm):
    def body(x_r, we_r, be_r, a_r, b_r, wm_r, bm_r, h_r, p_r, q_r, g_r):
        h = jnp.dot(x_r[...], we_r[...], preferred_element_type=_f32) + be_r[...]
        h_r[...] = h
        p_r[...] = jnp.dot(h, a_r[...], preferred_element_type=_f32)
        q_r[...] = jnp.dot(h, b_r[...], preferred_element_type=_f32)
        g_r[...] = jnp.dot(h, wm_r[...], preferred_element_type=_f32) + bm_r[...]
    return pl.pallas_call(
        body,
        compiler_params=pltpu.CompilerParams(skip_device_barrier=True),
        grid=(N // _BM_N,),
        in_specs=[_rows((_BM_N, EMB)), _full((EMB, EMB)), _full((1, EMB)),
                  _full((EMB, ED)), _full((EMB, ED)), _full((EMB, EMB)),
                  _full((1, EMB))],
        out_specs=[_rows((_BM_N, EMB)), _rows((_BM_N, ED)), _rows((_BM_N, ED)),
                   _rows((_BM_N, EMB))],
        out_shape=[jax.ShapeDtypeStruct((N, EMB), _f32),
                   jax.ShapeDtypeStruct((N, ED), _f32),
                   jax.ShapeDtypeStruct((N, ED), _f32),
                   jax.ShapeDtypeStruct((N, EMB), _f32)],
    )(x, We, be, A, B, Wm1, bm)


def _node_update_proj(h, pagg, Wu1, Wu2, bu, A, B, Wm1, bm):
    def body(h_r, pa_r, wu1_r, wu2_r, bu_r, a_r, b_r, wm_r, bm_r,
             hn_r, p_r, q_r, g_r):
        agg = pa_r[0] + pa_r[1]
        hn = jnp.maximum(
            jnp.dot(h_r[...], wu1_r[...], preferred_element_type=_f32)
            + jnp.dot(agg, wu2_r[...], preferred_element_type=_f32)
            + bu_r[...], 0.0)
        hn_r[...] = hn
        p_r[...] = jnp.dot(hn, a_r[...], preferred_element_type=_f32)
        q_r[...] = jnp.dot(hn, b_r[...], preferred_element_type=_f32)
        g_r[...] = jnp.dot(hn, wm_r[...], preferred_element_type=_f32) + bm_r[...]
    return pl.pallas_call(
        body,
        compiler_params=pltpu.CompilerParams(skip_device_barrier=True),
        grid=(N // _BM_N,),
        in_specs=[_rows((_BM_N, EMB)),
                  pl.BlockSpec((NC, _BM_N, EMB), lambda i: (0, i, 0)),
                  _full((EMB, EMB)), _full((EMB, EMB)), _full((1, EMB)),
                  _full((EMB, ED)), _full((EMB, ED)), _full((EMB, EMB)),
                  _full((1, EMB))],
        out_specs=[_rows((_BM_N, EMB)), _rows((_BM_N, ED)), _rows((_BM_N, ED)),
                   _rows((_BM_N, EMB))],
        out_shape=[jax.ShapeDtypeStruct((N, EMB), _f32),
                   jax.ShapeDtypeStruct((N, ED), _f32),
                   jax.ShapeDtypeStruct((N, ED), _f32),
                   jax.ShapeDtypeStruct((N, EMB), _f32)],
    )(h, pagg, Wu1, Wu2, bu, A, B, Wm1, bm)


def _node_final(h, pagg, Wu1, Wu2, bu, Wnp, bnp, WS, WD, bD):
    def body(h_r, pa_r, wu1_r, wu2_r, bu_r, wnp_r, bnp_r, ws_r, wd_r, bd_r,
             np_r, s_r, d_r):
        agg = pa_r[0] + pa_r[1]
        hn = jnp.maximum(
            jnp.dot(h_r[...], wu1_r[...], preferred_element_type=_f32)
            + jnp.dot(agg, wu2_r[...], preferred_element_type=_f32)
            + bu_r[...], 0.0)
        np_r[...] = jnp.dot(hn, wnp_r[...], preferred_element_type=_f32) + bnp_r[...]
        s_r[...] = jnp.dot(hn, ws_r[...], preferred_element_type=_f32)
        d_r[...] = jnp.dot(hn, wd_r[...], preferred_element_type=_f32) + bd_r[...]
    return pl.pallas_call(
        body,
        compiler_params=pltpu.CompilerParams(skip_device_barrier=True),
        grid=(N // _BM_N,),
        in_specs=[_rows((_BM_N, EMB)),
                  pl.BlockSpec((NC, _BM_N, EMB), lambda i: (0, i, 0)),
                  _full((EMB, EMB)), _full((EMB, EMB)), _full((1, EMB)),
                  _full((EMB, OUT_DIM)), _full((1, OUT_DIM)),
                  _full((EMB, 2 * ED)), _full((EMB, 2 * ED)), _full((1, 2 * ED))],
        out_specs=[_rows((_BM_N, OUT_DIM)), _rows((_BM_N, 2 * ED)),
                   _rows((_BM_N, 2 * ED))],
        out_shape=[jax.ShapeDtypeStruct((N, OUT_DIM), _f32),
                   jax.ShapeDtypeStruct((N, 2 * ED), _f32),
                   jax.ShapeDtypeStruct((N, 2 * ED), _f32)],
    )(h, pagg, Wu1, Wu2, bu, Wnp, bnp, WS, WD, bD)


def _edge_proj_first(ea, C, beu):
    def body(ea_r, c_r, b_r, out_r):
        out_r[...] = jnp.dot(ea_r[...], c_r[...],
                             preferred_element_type=_f32) + b_r[...]
    return pl.pallas_call(
        body,
        compiler_params=pltpu.CompilerParams(skip_device_barrier=True),
        grid=(E // _BM_E,),
        in_specs=[_rows((_BM_E, ED)), _full((ED, ED)), _full((1, ED))],
        out_specs=_rows((_BM_E, ED)),
        out_shape=jax.ShapeDtypeStruct((E, ED), _f32),
    )(ea, C, beu)


def _edge_proj(ea, Wm2, C, beu):
    def body(ea_r, wm_r, c_r, b_r, t_r, cc_r):
        v = ea_r[...]
        t_r[...] = jnp.dot(v, wm_r[...], preferred_element_type=_f32)
        cc_r[...] = jnp.dot(v, c_r[...], preferred_element_type=_f32) + b_r[...]
    return pl.pallas_call(
        body,
        compiler_params=pltpu.CompilerParams(skip_device_barrier=True),
        grid=(E // _BM_E,),
        in_specs=[_rows((_BM_E, ED)), _full((ED, EMB)), _full((ED, ED)),
                  _full((1, ED))],
        out_specs=[_rows((_BM_E, EMB)), _rows((_BM_E, ED))],
        out_shape=[jax.ShapeDtypeStruct((E, EMB), _f32),
                   jax.ShapeDtypeStruct((E, ED), _f32)],
    )(ea, Wm2, C, beu)


def _edge_head_sum(z):
    def body(z_r, out_r):
        out_r[...] = jnp.sum(z_r[...], axis=1, keepdims=True)
    return pl.pallas_call(
        body,
        compiler_params=pltpu.CompilerParams(skip_device_barrier=True),
        grid=(E // _BM_E,),
        in_specs=[_rows((_BM_E, ED))],
        out_specs=_rows((_BM_E, 1)),
        out_shape=jax.ShapeDtypeStruct((E, 1), _f32),
    )(z)


def _sc_params():
    return pltpu.CompilerParams(use_tc_tiling_on_sc=False,
                                skip_device_barrier=True)


def _chunk_off(base, ci):
    return pl.multiple_of(base + ci * CK, 8)


@functools.partial(
    pl.kernel,
    out_type=jax.ShapeDtypeStruct((E, ED), _f32),
    mesh=_mesh,
    compiler_params=_sc_params(),
    scratch_types=(
        [pltpu.VMEM((NCHUNK, CK), jnp.int32)] * 2
        + [pltpu.VMEM((CK, ED), _f32)] * (4 * NBUF)
        + [pltpu.SemaphoreType.DMA] * (4 * NBUF)
    ),
)
def _sc_edge_update(p_hbm, q_hbm, c_hbm, src3_hbm, dst3_hbm, out_hbm, *sc):
    srcb, dstb = sc[0], sc[1]
    PS = sc[2:2 + NBUF]
    QD = sc[2 + NBUF:2 + 2 * NBUF]
    CB = sc[2 + 2 * NBUF:2 + 3 * NBUF]
    OB = sc[2 + 3 * NBUF:2 + 4 * NBUF]
    base_s = 2 + 4 * NBUF
    SP = sc[base_s:base_s + NBUF]
    SQ = sc[base_s + NBUF:base_s + 2 * NBUF]
    SCM = sc[base_s + 2 * NBUF:base_s + 3 * NBUF]
    SW = sc[base_s + 3 * NBUF:base_s + 4 * NBUF]

    wid = lax.axis_index("c") * NS + lax.axis_index("s")
    base = wid * EPW
    pltpu.sync_copy(src3_hbm.at[wid], srcb)
    pltpu.sync_copy(dst3_hbm.at[wid], dstb)

    def issue(ci, b):
        off = _chunk_off(base, ci)
        pltpu.async_copy(p_hbm.at[srcb.at[ci]], PS[b], SP[b])
        pltpu.async_copy(q_hbm.at[dstb.at[ci]], QD[b], SQ[b])
        pltpu.async_copy(c_hbm.at[pl.ds(off, CK)], CB[b], SCM[b])

    def wait_wb(b, off):
        pltpu.make_async_copy(OB[b], out_hbm.at[pl.ds(off, CK)], SW[b]).wait()

    for b in range(NBUF):
        issue(b, b)

    def step(ci, b, wb_cond, issue_next):
        off = _chunk_off(base, ci)
        pltpu.make_async_copy(p_hbm.at[srcb.at[ci]], PS[b], SP[b]).wait()
        pltpu.make_async_copy(q_hbm.at[dstb.at[ci]], QD[b], SQ[b]).wait()
        pltpu.make_async_copy(c_hbm.at[pl.ds(off, CK)], CB[b], SCM[b]).wait()
        if wb_cond is True:
            wait_wb(b, off)
        elif wb_cond is not None:
            pl.when(wb_cond)(lambda: wait_wb(b, off))

        def ebody(k, c2):
            OB[b][k] = jnp.maximum(PS[b][k] + QD[b][k] + CB[b][k], 0.0)
            return c2
        lax.fori_loop(0, CK, ebody, 0)
        pltpu.async_copy(OB[b], out_hbm.at[pl.ds(off, CK)], SW[b])
        if issue_next:
            pl.when(ci + NBUF < NCHUNK)(lambda: issue(ci + NBUF, b))

    def outer(io, c2):
        for b in range(NBUF):
            step(NBUF * io + b, b, wb_cond=(io >= 1), issue_next=True)
        return c2

    lax.fori_loop(0, (NCHUNK - 1) // NBUF, outer, 0)
    step(NCHUNK - 1, (NCHUNK - 1) % NBUF, wb_cond=True, issue_next=False)
    for b in range(NBUF):
        wait_wb(b, base)


@functools.partial(
    pl.kernel,
    out_type=jax.ShapeDtypeStruct((E, 16), _f32),
    mesh=_mesh,
    compiler_params=_sc_params(),
    scratch_types=(
        [pltpu.VMEM((NCHUNK, CK), jnp.int32)] * 2
        + [pltpu.VMEM((CK, 2 * ED), _f32)] * (2 * NBUF)
        + [pltpu.VMEM((CK, ED), _f32)] * (2 * NBUF)
        + [pltpu.VMEM((16,), _f32)]
        + [pltpu.SemaphoreType.DMA] * (4 * NBUF)
    ),
)
def _sc_edge_pred(s_hbm, d_hbm, c_hbm, u3_hbm, src3_hbm, dst3_hbm, out_hbm,
                  *sc):
    srcb, dstb = sc[0], sc[1]
    PS = sc[2:2 + NBUF]
    QD = sc[2 + NBUF:2 + 2 * NBUF]
    CB = sc[2 + 2 * NBUF:2 + 3 * NBUF]
    OB = sc[2 + 3 * NBUF:2 + 4 * NBUF]
    u3v = sc[2 + 4 * NBUF]
    base_s = 3 + 4 * NBUF
    SP = sc[base_s:base_s + NBUF]
    SQ = sc[base_s + NBUF:base_s + 2 * NBUF]
    SCM = sc[base_s + 2 * NBUF:base_s + 3 * NBUF]
    SW = sc[base_s + 3 * NBUF:base_s + 4 * NBUF]

    wid = lax.axis_index("c") * NS + lax.axis_index("s")
    base = wid * EPW
    pltpu.sync_copy(src3_hbm.at[wid], srcb)
    pltpu.sync_copy(dst3_hbm.at[wid], dstb)
    pltpu.sync_copy(u3_hbm, u3v)

    def issue(ci, b):
        off = _chunk_off(base, ci)
        pltpu.async_copy(s_hbm.at[srcb.at[ci]], PS[b], SP[b])
        pltpu.async_copy(d_hbm.at[dstb.at[ci]], QD[b], SQ[b])
        pltpu.async_copy(c_hbm.at[pl.ds(off, CK)], CB[b], SCM[b])

    def wait_wb(b, off):
        pltpu.make_async_copy(OB[b], out_hbm.at[pl.ds(off, CK)], SW[b]).wait()

    for b in range(NBUF):
        issue(b, b)

    def step(ci, b, wb_cond, issue_next):
        off = _chunk_off(base, ci)
        pltpu.make_async_copy(s_hbm.at[srcb.at[ci]], PS[b], SP[b]).wait()
        pltpu.make_async_copy(d_hbm.at[dstb.at[ci]], QD[b], SQ[b]).wait()
        pltpu.make_async_copy(c_hbm.at[pl.ds(off, CK)], CB[b], SCM[b]).wait()
        if wb_cond is True:
            wait_wb(b, off)
        elif wb_cond is not None:
            pl.when(wb_cond)(lambda: wait_wb(b, off))

        def ebody(k, c2):
            pa = PS[b][k, pl.ds(0, ED)]
            pb = PS[b][k, pl.ds(ED, ED)]
            qa = QD[b][k, pl.ds(0, ED)]
            qb = QD[b][k, pl.ds(ED, ED)]
            ea2 = jnp.maximum(pa + qa + CB[b][k], 0.0)
            OB[b][k] = ea2 * u3v[...] + pb + qb
            return c2
        lax.fori_loop(0, CK, ebody, 0)
        pltpu.async_copy(OB[b], out_hbm.at[pl.ds(off, CK)], SW[b])
        if issue_next:
            pl.when(ci + NBUF < NCHUNK)(lambda: issue(ci + NBUF, b))

    def outer(io, c2):
        for b in range(NBUF):
            step(NBUF * io + b, b, wb_cond=(io >= 1), issue_next=True)
        return c2

    lax.fori_loop(0, (NCHUNK - 1) // NBUF, outer, 0)
    step(NCHUNK - 1, (NCHUNK - 1) % NBUF, wb_cond=True, issue_next=False)
    for b in range(NBUF):
        wait_wb(b, base)


@functools.partial(
    pl.kernel,
    out_type=jax.ShapeDtypeStruct((NC, N, EMB), _f32),
    mesh=_mesh,
    compiler_params=_sc_params(),
    scratch_types=(
        [pltpu.VMEM_SHARED((N, EMB), _f32)]
        + [pltpu.VMEM((ZROWS, EMB), _f32)]
        + [pltpu.VMEM((NCHM, CKM), jnp.int32)] * 2
        + [pltpu.VMEM((CKM, EMB), _f32)] * (2 * NBUFM)
        + [pltpu.SemaphoreType.DMA] * (2 * NBUFM)
    ),
)
def _sc_msg_agg(g_hbm, t_hbm, src3_hbm, dst3_hbm, out_hbm, *sc):
    agg_sh, zb = sc[0], sc[1]
    srcb, dstb = sc[2], sc[3]
    ROWS = sc[4:4 + NBUFM]
    TB = sc[4 + NBUFM:4 + 2 * NBUFM]
    SG = sc[4 + 2 * NBUFM:4 + 3 * NBUFM]
    ST = sc[4 + 3 * NBUFM:4 + 4 * NBUFM]

    cid = lax.axis_index("c")
    sid = lax.axis_index("s")
    wid = cid * NS + sid
    base = wid * EPW
    pltpu.sync_copy(src3_hbm.at[wid], srcb)
    pltpu.sync_copy(dst3_hbm.at[wid], dstb)

    def zrow(k, carry):
        for j in range(EMB // 16):
            zb[k, pl.ds(j * 16, 16)] = jnp.zeros((16,), _f32)
        return carry
    lax.fori_loop(0, ZROWS, zrow, 0)

    def zcp(i, carry):
        pltpu.sync_copy(zb, agg_sh.at[pl.ds(sid * RPT + i * ZROWS, ZROWS)])
        return carry
    lax.fori_loop(0, RPT // ZROWS, zcp, 0)
    plsc.subcore_barrier()

    def issue(ci, b):
        off = base + ci * CKM
        pltpu.async_copy(g_hbm.at[dstb.at[ci]], ROWS[b], SG[b])
        pltpu.async_copy(t_hbm.at[pl.ds(pl.multiple_of(off, 8), CKM)],
                         TB[b], ST[b])

    for b in range(NBUFM):
        issue(b, b)

    def step(ci, b):
        off = base + ci * CKM
        pltpu.make_async_copy(g_hbm.at[dstb.at[ci]], ROWS[b], SG[b]).wait()
        pltpu.make_async_copy(t_hbm.at[pl.ds(pl.multiple_of(off, 8), CKM)],
                              TB[b], ST[b]).wait()

        def ebody(k, c2):
            for j in range(EMB // 16):
                sl = pl.ds(j * 16, 16)
                ROWS[b][k, sl] = jnp.maximum(ROWS[b][k, sl] + TB[b][k, sl], 0.0)
            return c2
        lax.fori_loop(0, CKM, ebody, 0)
        pltpu.sync_copy(ROWS[b], agg_sh.at[srcb.at[ci]], add=True)
        pl.when(ci + NBUFM < NCHM)(lambda: issue(ci + NBUFM, b))

    def outer(io, c2):
        for b in range(NBUFM):
            step(NBUFM * io + b, b)
        return c2

    lax.fori_loop(0, NCHM // NBUFM, outer, 0)
    plsc.subcore_barrier()
    pltpu.sync_copy(agg_sh.at[pl.ds(sid * RPT, RPT)],
                    out_hbm.at[cid, pl.ds(sid * RPT, RPT)])


def kernel(x, edge_attr, edge_index, W_emb, b_emb, W_msg, b_msg, W_upd, b_upd,
           W_eu, b_eu, W_np, b_np, W_ep, b_ep):
    src = edge_index[0]
    dst = edge_index[1]
    src3 = src.reshape(NW, NCHUNK, CK)
    dst3 = dst.reshape(NW, NCHUNK, CK)
    srcm = src.reshape(NW, NCHM, CKM)
    dstm = dst.reshape(NW, NCHM, CKM)

    def r2(b):
        return b.reshape(1, -1)

    A = [W_eu[l][:EMB] for l in range(3)]
    B = [W_eu[l][EMB:2 * EMB] for l in range(3)]
    C = [W_eu[l][2 * EMB:] for l in range(3)]
    Wm1 = [W_msg[l][:EMB] for l in range(2)]
    Wm2 = [W_msg[l][EMB:] for l in range(2)]
    Wu1 = [W_upd[l][:EMB] for l in range(2)]
    Wu2 = [W_upd[l][EMB:] for l in range(2)]

    zpad = jnp.zeros((EMB, ED - 1), _f32)
    WS = jnp.concatenate([A[2], W_ep[:EMB], zpad], axis=1)
    WD = jnp.concatenate([B[2], W_ep[EMB:2 * EMB], zpad], axis=1)
    bD = jnp.zeros((2 * ED,), _f32).at[ED + 1].set(b_ep[0])
    u3 = W_ep[2 * EMB:, 0]

    h0, P0, Q0, g0 = _node_emb_proj(x, W_emb, r2(b_emb), A[0], B[0],
                                    Wm1[0], r2(b_msg[0]))
    c0 = _edge_proj_first(edge_attr, C[0], r2(b_eu[0]))
    ea0 = _sc_edge_update(P0, Q0, c0, src3, dst3)
    t0, c1 = _edge_proj(ea0, Wm2[0], C[1], r2(b_eu[1]))
    pagg0 = _sc_msg_agg(g0, t0, srcm, dstm)

    h1, P1, Q1, g1 = _node_update_proj(h0, pagg0, Wu1[0], Wu2[0],
                                       r2(b_upd[0]), A[1], B[1],
                                       Wm1[1], r2(b_msg[1]))
    ea1 = _sc_edge_update(P1, Q1, c1, src3, dst3)
    t1, c2 = _edge_proj(ea1, Wm2[1], C[2], r2(b_eu[2]))
    pagg1 = _sc_msg_agg(g1, t1, srcm, dstm)

    npred, S2, D2 = _node_final(h1, pagg1, Wu1[1], Wu2[1], r2(b_upd[1]),
                                W_np, r2(b_np), WS, WD, r2(bD))
    z16 = _sc_edge_pred(S2, D2, c2, u3, src3, dst3)
    return (npred, _edge_head_sum(z16))


# speedup vs baseline: 4.7822x; 1.6105x over previous
import functools

import jax
import jax.numpy as jnp
from jax import lax
from jax.experimental import pallas as pl
from jax.experimental.pallas import tpu as pltpu
from jax.experimental.pallas import tpu_sc as plsc

N = 10000
E = 320000
EMB = 128
ED = 16
OUT_DIM = 128

NC = 2
NS = 16
NW = NC * NS
EPW = E // NW
CK = 80
NCHUNK = EPW // CK
NBUF = 4
CKM = 40
NCHM = EPW // CKM
NBUFM = 2
RPT = N // NS
ZROWS = 25

PK = 8
RPW = EPW // PK
CKR = CK // PK
CKMR = CKM // PK

_mesh = plsc.VectorSubcoreMesh(
    core_axis_name="c", subcore_axis_name="s", num_cores=NC, num_subcores=NS)

_f32 = jnp.float32


_BM_N = 2000
_BM_E = 2000


def _full(shape):
    return pl.BlockSpec(shape, lambda i: (0,) * len(shape))


def _rows(shape):
    return pl.BlockSpec(shape, lambda i: (i,) + (0,) * (len(shape) - 1))


def _node_emb_proj(x, We, be, A, B, Wm1, bm):
    def body(x_r, we_r, be_r, a_r, b_r, wm_r, bm_r, h_r, p_r, q_r, g_r):
        h = jnp.dot(x_r[...], we_r[...], preferred_element_type=_f32) + be_r[...]
        h_r[...] = h
        p_r[...] = jnp.dot(h, a_r[...], preferred_element_type=_f32)
        q_r[...] = jnp.dot(h, b_r[...], preferred_element_type=_f32)
        g_r[...] = jnp.dot(h, wm_r[...], preferred_element_type=_f32) + bm_r[...]
    return pl.pallas_call(
        body,
        compiler_params=pltpu.CompilerParams(skip_device_barrier=True),
        grid=(N // _BM_N,),
        in_specs=[_rows((_BM_N, EMB)), _full((EMB, EMB)), _full((1, EMB)),
                  _full((EMB, ED)), _full((EMB, ED)), _full((EMB, EMB)),
                  _full((1, EMB))],
        out_specs=[_rows((_BM_N, EMB)), _rows((_BM_N, ED)), _rows((_BM_N, ED)),
                   _rows((_BM_N, EMB))],
        out_shape=[jax.ShapeDtypeStruct((N, EMB), _f32),
                   jax.ShapeDtypeStruct((N, ED), _f32),
                   jax.ShapeDtypeStruct((N, ED), _f32),
                   jax.ShapeDtypeStruct((N, EMB), _f32)],
    )(x, We, be, A, B, Wm1, bm)


def _node_update_proj(h, pagg, Wu1, Wu2, bu, A, B, Wm1, bm):
    def body(h_r, pa_r, wu1_r, wu2_r, bu_r, a_r, b_r, wm_r, bm_r,
             hn_r, p_r, q_r, g_r):
        agg = pa_r[0] + pa_r[1]
        hn = jnp.maximum(
            jnp.dot(h_r[...], wu1_r[...], preferred_element_type=_f32)
            + jnp.dot(agg, wu2_r[...], preferred_element_type=_f32)
            + bu_r[...], 0.0)
        hn_r[...] = hn
        p_r[...] = jnp.dot(hn, a_r[...], preferred_element_type=_f32)
        q_r[...] = jnp.dot(hn, b_r[...], preferred_element_type=_f32)
        g_r[...] = jnp.dot(hn, wm_r[...], preferred_element_type=_f32) + bm_r[...]
    return pl.pallas_call(
        body,
        compiler_params=pltpu.CompilerParams(skip_device_barrier=True),
        grid=(N // _BM_N,),
        in_specs=[_rows((_BM_N, EMB)),
                  pl.BlockSpec((NC, _BM_N, EMB), lambda i: (0, i, 0)),
                  _full((EMB, EMB)), _full((EMB, EMB)), _full((1, EMB)),
                  _full((EMB, ED)), _full((EMB, ED)), _full((EMB, EMB)),
                  _full((1, EMB))],
        out_specs=[_rows((_BM_N, EMB)), _rows((_BM_N, ED)), _rows((_BM_N, ED)),
                   _rows((_BM_N, EMB))],
        out_shape=[jax.ShapeDtypeStruct((N, EMB), _f32),
                   jax.ShapeDtypeStruct((N, ED), _f32),
                   jax.ShapeDtypeStruct((N, ED), _f32),
                   jax.ShapeDtypeStruct((N, EMB), _f32)],
    )(h, pagg, Wu1, Wu2, bu, A, B, Wm1, bm)


def _node_final(h, pagg, Wu1, Wu2, bu, Wnp, bnp, WS, WD, bD):
    def body(h_r, pa_r, wu1_r, wu2_r, bu_r, wnp_r, bnp_r, ws_r, wd_r, bd_r,
             np_r, s_r, d_r):
        agg = pa_r[0] + pa_r[1]
        hn = jnp.maximum(
            jnp.dot(h_r[...], wu1_r[...], preferred_element_type=_f32)
            + jnp.dot(agg, wu2_r[...], preferred_element_type=_f32)
            + bu_r[...], 0.0)
        np_r[...] = jnp.dot(hn, wnp_r[...], preferred_element_type=_f32) + bnp_r[...]
        s_r[...] = jnp.dot(hn, ws_r[...], preferred_element_type=_f32)
        d_r[...] = jnp.dot(hn, wd_r[...], preferred_element_type=_f32) + bd_r[...]
    return pl.pallas_call(
        body,
        compiler_params=pltpu.CompilerParams(skip_device_barrier=True),
        grid=(N // _BM_N,),
        in_specs=[_rows((_BM_N, EMB)),
                  pl.BlockSpec((NC, _BM_N, EMB), lambda i: (0, i, 0)),
                  _full((EMB, EMB)), _full((EMB, EMB)), _full((1, EMB)),
                  _full((EMB, OUT_DIM)), _full((1, OUT_DIM)),
                  _full((EMB, 2 * ED)), _full((EMB, 2 * ED)), _full((1, 2 * ED))],
        out_specs=[_rows((_BM_N, OUT_DIM)), _rows((_BM_N, 2 * ED)),
                   _rows((_BM_N, 2 * ED))],
        out_shape=[jax.ShapeDtypeStruct((N, OUT_DIM), _f32),
                   jax.ShapeDtypeStruct((N, 2 * ED), _f32),
                   jax.ShapeDtypeStruct((N, 2 * ED), _f32)],
    )(h, pagg, Wu1, Wu2, bu, Wnp, bnp, WS, WD, bD)


_BM_P = 2000


def _edge_proj_first(eap, BDC, btile):
    def body(ea_r, c_r, b_r, out_r):
        out_r[...] = jnp.dot(ea_r[...], c_r[...],
                             preferred_element_type=_f32) + b_r[...]
    return pl.pallas_call(
        body,
        compiler_params=pltpu.CompilerParams(skip_device_barrier=True),
        grid=(E // PK // _BM_P,),
        in_specs=[_rows((_BM_P, 128)), _full((128, 128)), _full((1, 128))],
        out_specs=_rows((_BM_P, 128)),
        out_shape=jax.ShapeDtypeStruct((E // PK, 128), _f32),
    )(eap, BDC, btile)


def _edge_proj(eap, BDW, BDC, btile):
    def body(ea_r, wm_r, c_r, b_r, t_r, cc_r):
        v = ea_r[...]
        t_r[...] = jnp.dot(v, wm_r[...], preferred_element_type=_f32)
        cc_r[...] = jnp.dot(v, c_r[...], preferred_element_type=_f32) + b_r[...]
    return pl.pallas_call(
        body,
        compiler_params=pltpu.CompilerParams(skip_device_barrier=True),
        grid=(E // PK // _BM_P,),
        in_specs=[_rows((_BM_P, 128)), _full((128, PK * EMB)),
                  _full((128, 128)), _full((1, 128))],
        out_specs=[_rows((_BM_P, PK * EMB)), _rows((_BM_P, 128))],
        out_shape=[jax.ShapeDtypeStruct((E // PK, PK * EMB), _f32),
                   jax.ShapeDtypeStruct((E // PK, 128), _f32)],
    )(eap, BDW, BDC, btile)


def _edge_head_sum(zp, BDones):
    def body(z_r, w_r, out_r):
        out_r[...] = jnp.dot(z_r[...], w_r[...], preferred_element_type=_f32)
    return pl.pallas_call(
        body,
        compiler_params=pltpu.CompilerParams(skip_device_barrier=True),
        grid=(E // PK // _BM_P,),
        in_specs=[_rows((_BM_P, 128)), _full((128, PK))],
        out_specs=_rows((_BM_P, PK)),
        out_shape=jax.ShapeDtypeStruct((E // PK, PK), _f32),
    )(zp, BDones)


def _sc_params():
    return pltpu.CompilerParams(use_tc_tiling_on_sc=False,
                                skip_device_barrier=True)


def _chunk_off(base, ci):
    return pl.multiple_of(base + ci * CK, 8)


@functools.partial(
    pl.kernel,
    out_type=jax.ShapeDtypeStruct((E // PK, 128), _f32),
    mesh=_mesh,
    compiler_params=_sc_params(),
    scratch_types=(
        [pltpu.VMEM((NCHUNK, CK), jnp.int32)] * 2
        + [pltpu.VMEM((CK, ED), _f32)] * (2 * NBUF)
        + [pltpu.VMEM((CKR, 128), _f32)] * (2 * NBUF)
        + [pltpu.SemaphoreType.DMA] * (4 * NBUF)
    ),
)
def _sc_edge_update(p_hbm, q_hbm, c_hbm, src3_hbm, dst3_hbm, out_hbm, *sc):
    srcb, dstb = sc[0], sc[1]
    PS = sc[2:2 + NBUF]
    QD = sc[2 + NBUF:2 + 2 * NBUF]
    CB = sc[2 + 2 * NBUF:2 + 3 * NBUF]
    OB = sc[2 + 3 * NBUF:2 + 4 * NBUF]
    base_s = 2 + 4 * NBUF
    SP = sc[base_s:base_s + NBUF]
    SQ = sc[base_s + NBUF:base_s + 2 * NBUF]
    SCM = sc[base_s + 2 * NBUF:base_s + 3 * NBUF]
    SW = sc[base_s + 3 * NBUF:base_s + 4 * NBUF]

    wid = lax.axis_index("c") * NS + lax.axis_index("s")
    base = wid * EPW
    rbase = wid * RPW
    pltpu.sync_copy(src3_hbm.at[wid], srcb)
    pltpu.sync_copy(dst3_hbm.at[wid], dstb)

    def issue(ci, b):
        roff = rbase + ci * CKR
        pltpu.async_copy(p_hbm.at[srcb.at[ci]], PS[b], SP[b])
        pltpu.async_copy(q_hbm.at[dstb.at[ci]], QD[b], SQ[b])
        pltpu.async_copy(c_hbm.at[pl.ds(roff, CKR)], CB[b], SCM[b])

    def wait_wb(b, roff):
        pltpu.make_async_copy(OB[b], out_hbm.at[pl.ds(roff, CKR)], SW[b]).wait()

    for b in range(NBUF):
        issue(b, b)

    def step(ci, b, wb_cond, issue_next):
        roff = rbase + ci * CKR
        pltpu.make_async_copy(p_hbm.at[srcb.at[ci]], PS[b], SP[b]).wait()
        pltpu.make_async_copy(q_hbm.at[dstb.at[ci]], QD[b], SQ[b]).wait()
        pltpu.make_async_copy(c_hbm.at[pl.ds(roff, CKR)], CB[b], SCM[b]).wait()
        if wb_cond is True:
            wait_wb(b, roff)
        elif wb_cond is not None:
            pl.when(wb_cond)(lambda: wait_wb(b, roff))

        def ebody(k8, c2):
            for e in range(PK):
                sl = pl.ds(e * ED, ED)
                OB[b][k8, sl] = jnp.maximum(
                    PS[b][k8 * PK + e] + QD[b][k8 * PK + e] + CB[b][k8, sl],
                    0.0)
            return c2
        lax.fori_loop(0, CKR, ebody, 0)
        pltpu.async_copy(OB[b], out_hbm.at[pl.ds(roff, CKR)], SW[b])
        if issue_next:
            pl.when(ci + NBUF < NCHUNK)(lambda: issue(ci + NBUF, b))

    def outer(io, c2):
        for b in range(NBUF):
            step(NBUF * io + b, b, wb_cond=(io >= 1), issue_next=True)
        return c2

    lax.fori_loop(0, (NCHUNK - 1) // NBUF, outer, 0)
    step(NCHUNK - 1, (NCHUNK - 1) % NBUF, wb_cond=True, issue_next=False)
    for b in range(NBUF):
        wait_wb(b, rbase)


@functools.partial(
    pl.kernel,
    out_type=jax.ShapeDtypeStruct((E // PK, 128), _f32),
    mesh=_mesh,
    compiler_params=_sc_params(),
    scratch_types=(
        [pltpu.VMEM((NCHUNK, CK), jnp.int32)] * 2
        + [pltpu.VMEM((CK, 2 * ED), _f32)] * (2 * NBUF)
        + [pltpu.VMEM((CKR, 128), _f32)] * (2 * NBUF)
        + [pltpu.VMEM((16,), _f32)]
        + [pltpu.SemaphoreType.DMA] * (4 * NBUF)
    ),
)
def _sc_edge_pred(s_hbm, d_hbm, c_hbm, u3_hbm, src3_hbm, dst3_hbm, out_hbm,
                  *sc):
    srcb, dstb = sc[0], sc[1]
    PS = sc[2:2 + NBUF]
    QD = sc[2 + NBUF:2 + 2 * NBUF]
    CB = sc[2 + 2 * NBUF:2 + 3 * NBUF]
    OB = sc[2 + 3 * NBUF:2 + 4 * NBUF]
    u3v = sc[2 + 4 * NBUF]
    base_s = 3 + 4 * NBUF
    SP = sc[base_s:base_s + NBUF]
    SQ = sc[base_s + NBUF:base_s + 2 * NBUF]
    SCM = sc[base_s + 2 * NBUF:base_s + 3 * NBUF]
    SW = sc[base_s + 3 * NBUF:base_s + 4 * NBUF]

    wid = lax.axis_index("c") * NS + lax.axis_index("s")
    rbase = wid * RPW
    pltpu.sync_copy(src3_hbm.at[wid], srcb)
    pltpu.sync_copy(dst3_hbm.at[wid], dstb)
    pltpu.sync_copy(u3_hbm, u3v)

    def issue(ci, b):
        roff = rbase + ci * CKR
        pltpu.async_copy(s_hbm.at[srcb.at[ci]], PS[b], SP[b])
        pltpu.async_copy(d_hbm.at[dstb.at[ci]], QD[b], SQ[b])
        pltpu.async_copy(c_hbm.at[pl.ds(roff, CKR)], CB[b], SCM[b])

    def wait_wb(b, roff):
        pltpu.make_async_copy(OB[b], out_hbm.at[pl.ds(roff, CKR)], SW[b]).wait()

    for b in range(NBUF):
        issue(b, b)

    def step(ci, b, wb_cond, issue_next):
        roff = rbase + ci * CKR
        pltpu.make_async_copy(s_hbm.at[srcb.at[ci]], PS[b], SP[b]).wait()
        pltpu.make_async_copy(d_hbm.at[dstb.at[ci]], QD[b], SQ[b]).wait()
        pltpu.make_async_copy(c_hbm.at[pl.ds(roff, CKR)], CB[b], SCM[b]).wait()
        if wb_cond is True:
            wait_wb(b, roff)
        elif wb_cond is not None:
            pl.when(wb_cond)(lambda: wait_wb(b, roff))

        def ebody(k8, c2):
            for e in range(PK):
                k = k8 * PK + e
                sl = pl.ds(e * ED, ED)
                pa = PS[b][k, pl.ds(0, ED)]
                pb = PS[b][k, pl.ds(ED, ED)]
                qa = QD[b][k, pl.ds(0, ED)]
                qb = QD[b][k, pl.ds(ED, ED)]
                ea2 = jnp.maximum(pa + qa + CB[b][k8, sl], 0.0)
                OB[b][k8, sl] = ea2 * u3v[...] + pb + qb
            return c2
        lax.fori_loop(0, CKR, ebody, 0)
        pltpu.async_copy(OB[b], out_hbm.at[pl.ds(roff, CKR)], SW[b])
        if issue_next:
            pl.when(ci + NBUF < NCHUNK)(lambda: issue(ci + NBUF, b))

    def outer(io, c2):
        for b in range(NBUF):
            step(NBUF * io + b, b, wb_cond=(io >= 1), issue_next=True)
        return c2

    lax.fori_loop(0, (NCHUNK - 1) // NBUF, outer, 0)
    step(NCHUNK - 1, (NCHUNK - 1) % NBUF, wb_cond=True, issue_next=False)
    for b in range(NBUF):
        wait_wb(b, rbase)


@functools.partial(
    pl.kernel,
    out_type=jax.ShapeDtypeStruct((NC, N, EMB), _f32),
    mesh=_mesh,
    compiler_params=_sc_params(),
    scratch_types=(
        [pltpu.VMEM_SHARED((N, EMB), _f32)]
        + [pltpu.VMEM((ZROWS, EMB), _f32)]
        + [pltpu.VMEM((NCHM, CKM), jnp.int32)] * 2
        + [pltpu.VMEM((CKM, EMB), _f32)] * NBUFM
        + [pltpu.VMEM((CKMR, PK * EMB), _f32)] * NBUFM
        + [pltpu.SemaphoreType.DMA] * (2 * NBUFM)
    ),
)
def _sc_msg_agg(g_hbm, t_hbm, src3_hbm, dst3_hbm, out_hbm, *sc):
    agg_sh, zb = sc[0], sc[1]
    srcb, dstb = sc[2], sc[3]
    ROWS = sc[4:4 + NBUFM]
    TB = sc[4 + NBUFM:4 + 2 * NBUFM]
    SG = sc[4 + 2 * NBUFM:4 + 3 * NBUFM]
    ST = sc[4 + 3 * NBUFM:4 + 4 * NBUFM]

    cid = lax.axis_index("c")
    sid = lax.axis_index("s")
    wid = cid * NS + sid
    rbase = wid * RPW
    pltpu.sync_copy(src3_hbm.at[wid], srcb)
    pltpu.sync_copy(dst3_hbm.at[wid], dstb)

    def zrow(k, carry):
        for j in range(EMB // 16):
            zb[k, pl.ds(j * 16, 16)] = jnp.zeros((16,), _f32)
        return carry
    lax.fori_loop(0, ZROWS, zrow, 0)

    def zcp(i, carry):
        pltpu.sync_copy(zb, agg_sh.at[pl.ds(sid * RPT + i * ZROWS, ZROWS)])
        return carry
    lax.fori_loop(0, RPT // ZROWS, zcp, 0)
    plsc.subcore_barrier()

    def issue(ci, b):
        roff = rbase + ci * CKMR
        pltpu.async_copy(g_hbm.at[dstb.at[ci]], ROWS[b], SG[b])
        pltpu.async_copy(t_hbm.at[pl.ds(roff, CKMR)], TB[b], ST[b])

    for b in range(NBUFM):
        issue(b, b)

    def step(ci, b):
        roff = rbase + ci * CKMR
        pltpu.make_async_copy(g_hbm.at[dstb.at[ci]], ROWS[b], SG[b]).wait()
        pltpu.make_async_copy(t_hbm.at[pl.ds(roff, CKMR)], TB[b], ST[b]).wait()

        def ebody(k8, c2):
            for e in range(PK):
                for j in range(EMB // 16):
                    sl = pl.ds(j * 16, 16)
                    tsl = pl.ds(e * EMB + j * 16, 16)
                    ROWS[b][k8 * PK + e, sl] = jnp.maximum(
                        ROWS[b][k8 * PK + e, sl] + TB[b][k8, tsl], 0.0)
            return c2
        lax.fori_loop(0, CKMR, ebody, 0)
        pltpu.sync_copy(ROWS[b], agg_sh.at[srcb.at[ci]], add=True)
        pl.when(ci + NBUFM < NCHM)(lambda: issue(ci + NBUFM, b))

    def outer(io, c2):
        for b in range(NBUFM):
            step(NBUFM * io + b, b)
        return c2

    lax.fori_loop(0, NCHM // NBUFM, outer, 0)
    plsc.subcore_barrier()
    pltpu.sync_copy(agg_sh.at[pl.ds(sid * RPT, RPT)],
                    out_hbm.at[cid, pl.ds(sid * RPT, RPT)])


def kernel(x, edge_attr, edge_index, W_emb, b_emb, W_msg, b_msg, W_upd, b_upd,
           W_eu, b_eu, W_np, b_np, W_ep, b_ep):
    src = edge_index[0]
    dst = edge_index[1]
    src3 = src.reshape(NW, NCHUNK, CK)
    dst3 = dst.reshape(NW, NCHUNK, CK)
    srcm = src.reshape(NW, NCHM, CKM)
    dstm = dst.reshape(NW, NCHM, CKM)

    def r2(b):
        return b.reshape(1, -1)

    A = [W_eu[l][:EMB] for l in range(3)]
    B = [W_eu[l][EMB:2 * EMB] for l in range(3)]
    C = [W_eu[l][2 * EMB:] for l in range(3)]
    Wm1 = [W_msg[l][:EMB] for l in range(2)]
    Wm2 = [W_msg[l][EMB:] for l in range(2)]
    Wu1 = [W_upd[l][:EMB] for l in range(2)]
    Wu2 = [W_upd[l][EMB:] for l in range(2)]

    zpad = jnp.zeros((EMB, ED - 1), _f32)
    WS = jnp.concatenate([A[2], W_ep[:EMB], zpad], axis=1)
    WD = jnp.concatenate([B[2], W_ep[EMB:2 * EMB], zpad], axis=1)
    bD = jnp.zeros((2 * ED,), _f32).at[ED + 1].set(b_ep[0])
    u3 = W_ep[2 * EMB:, 0]

    eye8 = jnp.eye(PK, dtype=_f32)
    bdC = [jnp.kron(eye8, C[l]) for l in range(3)]
    bdW = [jnp.kron(eye8, Wm2[l]) for l in range(2)]
    btile = [r2(jnp.tile(b_eu[l], PK)) for l in range(3)]
    bdones = jnp.kron(eye8, jnp.ones((ED, 1), _f32))
    eap = edge_attr.reshape(E // PK, PK * ED)

    h0, P0, Q0, g0 = _node_emb_proj(x, W_emb, r2(b_emb), A[0], B[0],
                                    Wm1[0], r2(b_msg[0]))
    c0 = _edge_proj_first(eap, bdC[0], btile[0])
    ea0 = _sc_edge_update(P0, Q0, c0, src3, dst3)
    t0, c1 = _edge_proj(ea0, bdW[0], bdC[1], btile[1])
    pagg0 = _sc_msg_agg(g0, t0, srcm, dstm)

    h1, P1, Q1, g1 = _node_update_proj(h0, pagg0, Wu1[0], Wu2[0],
                                       r2(b_upd[0]), A[1], B[1],
                                       Wm1[1], r2(b_msg[1]))
    ea1 = _sc_edge_update(P1, Q1, c1, src3, dst3)
    t1, c2 = _edge_proj(ea1, bdW[1], bdC[2], btile[2])
    pagg1 = _sc_msg_agg(g1, t1, srcm, dstm)

    npred, S2, D2 = _node_final(h1, pagg1, Wu1[1], Wu2[1], r2(b_upd[1]),
                                W_np, r2(b_np), WS, WD, r2(bD))
    z16 = _sc_edge_pred(S2, D2, c2, u3, src3, dst3)
    return (npred, _edge_head_sum(z16, bdones).reshape(E, 1))


# speedup vs baseline: 5.8626x; 1.2259x over previous
import functools

import jax
import jax.numpy as jnp
from jax import lax
from jax.experimental import pallas as pl
from jax.experimental.pallas import tpu as pltpu
from jax.experimental.pallas import tpu_sc as plsc

N = 10000
E = 320000
EMB = 128
ED = 16
OUT_DIM = 128

NC = 2
NS = 16
NW = NC * NS
EPW = E // NW
CK = 80
NCHUNK = EPW // CK
NBUF = 4
CKM = 40
NCHM = EPW // CKM
NBUFM = 2
RPT = N // NS
ZROWS = 25

PK = 8
RPW = EPW // PK
CKR = CK // PK
CKMR = CKM // PK

_mesh = plsc.VectorSubcoreMesh(
    core_axis_name="c", subcore_axis_name="s", num_cores=NC, num_subcores=NS)

_f32 = jnp.float32


_BM_N = 2000
_BM_E = 2000


def _full(shape):
    return pl.BlockSpec(shape, lambda i: (0,) * len(shape))


def _rows(shape):
    return pl.BlockSpec(shape, lambda i: (i,) + (0,) * (len(shape) - 1))


def _node_emb_proj(x, We, be, A, B, Wm1, bm):
    def body(x_r, we_r, be_r, a_r, b_r, wm_r, bm_r, h_r, p_r, q_r, g_r):
        h = jnp.dot(x_r[...], we_r[...], preferred_element_type=_f32) + be_r[...]
        h_r[...] = h
        p_r[...] = jnp.dot(h, a_r[...], preferred_element_type=_f32)
        q_r[...] = jnp.dot(h, b_r[...], preferred_element_type=_f32)
        g_r[...] = jnp.dot(h, wm_r[...], preferred_element_type=_f32) + bm_r[...]
    return pl.pallas_call(
        body,
        compiler_params=pltpu.CompilerParams(skip_device_barrier=True),
        grid=(N // _BM_N,),
        in_specs=[_rows((_BM_N, EMB)), _full((EMB, EMB)), _full((1, EMB)),
                  _full((EMB, ED)), _full((EMB, ED)), _full((EMB, EMB)),
                  _full((1, EMB))],
        out_specs=[_rows((_BM_N, EMB)), _rows((_BM_N, ED)), _rows((_BM_N, ED)),
                   _rows((_BM_N, EMB))],
        out_shape=[jax.ShapeDtypeStruct((N, EMB), _f32),
                   jax.ShapeDtypeStruct((N, ED), _f32),
                   jax.ShapeDtypeStruct((N, ED), _f32),
                   jax.ShapeDtypeStruct((N, EMB), _f32)],
    )(x, We, be, A, B, Wm1, bm)


def _node_update_proj(h, pagg, Wu1, Wu2, bu, A, B, Wm1, bm):
    def body(h_r, pa_r, wu1_r, wu2_r, bu_r, a_r, b_r, wm_r, bm_r,
             hn_r, p_r, q_r, g_r):
        agg = pa_r[0] + pa_r[1]
        hn = jnp.maximum(
            jnp.dot(h_r[...], wu1_r[...], preferred_element_type=_f32)
            + jnp.dot(agg, wu2_r[...], preferred_element_type=_f32)
            + bu_r[...], 0.0)
        hn_r[...] = hn
        p_r[...] = jnp.dot(hn, a_r[...], preferred_element_type=_f32)
        q_r[...] = jnp.dot(hn, b_r[...], preferred_element_type=_f32)
        g_r[...] = jnp.dot(hn, wm_r[...], preferred_element_type=_f32) + bm_r[...]
    return pl.pallas_call(
        body,
        compiler_params=pltpu.CompilerParams(skip_device_barrier=True),
        grid=(N // _BM_N,),
        in_specs=[_rows((_BM_N, EMB)),
                  pl.BlockSpec((NC, _BM_N, EMB), lambda i: (0, i, 0)),
                  _full((EMB, EMB)), _full((EMB, EMB)), _full((1, EMB)),
                  _full((EMB, ED)), _full((EMB, ED)), _full((EMB, EMB)),
                  _full((1, EMB))],
        out_specs=[_rows((_BM_N, EMB)), _rows((_BM_N, ED)), _rows((_BM_N, ED)),
                   _rows((_BM_N, EMB))],
        out_shape=[jax.ShapeDtypeStruct((N, EMB), _f32),
                   jax.ShapeDtypeStruct((N, ED), _f32),
                   jax.ShapeDtypeStruct((N, ED), _f32),
                   jax.ShapeDtypeStruct((N, EMB), _f32)],
    )(h, pagg, Wu1, Wu2, bu, A, B, Wm1, bm)


def _node_final(h, pagg, Wu1, Wu2, bu, Wnp, bnp, WS, WD, bD):
    def body(h_r, pa_r, wu1_r, wu2_r, bu_r, wnp_r, bnp_r, ws_r, wd_r, bd_r,
             np_r, s_r, d_r):
        agg = pa_r[0] + pa_r[1]
        hn = jnp.maximum(
            jnp.dot(h_r[...], wu1_r[...], preferred_element_type=_f32)
            + jnp.dot(agg, wu2_r[...], preferred_element_type=_f32)
            + bu_r[...], 0.0)
        np_r[...] = jnp.dot(hn, wnp_r[...], preferred_element_type=_f32) + bnp_r[...]
        s_r[...] = jnp.dot(hn, ws_r[...], preferred_element_type=_f32)
        d_r[...] = jnp.dot(hn, wd_r[...], preferred_element_type=_f32) + bd_r[...]
    return pl.pallas_call(
        body,
        compiler_params=pltpu.CompilerParams(skip_device_barrier=True),
        grid=(N // _BM_N,),
        in_specs=[_rows((_BM_N, EMB)),
                  pl.BlockSpec((NC, _BM_N, EMB), lambda i: (0, i, 0)),
                  _full((EMB, EMB)), _full((EMB, EMB)), _full((1, EMB)),
                  _full((EMB, OUT_DIM)), _full((1, OUT_DIM)),
                  _full((EMB, 2 * ED)), _full((EMB, 2 * ED)), _full((1, 2 * ED))],
        out_specs=[_rows((_BM_N, OUT_DIM)), _rows((_BM_N, 2 * ED)),
                   _rows((_BM_N, 2 * ED))],
        out_shape=[jax.ShapeDtypeStruct((N, OUT_DIM), _f32),
                   jax.ShapeDtypeStruct((N, 2 * ED), _f32),
                   jax.ShapeDtypeStruct((N, 2 * ED), _f32)],
    )(h, pagg, Wu1, Wu2, bu, Wnp, bnp, WS, WD, bD)


_BM_P = 2000


def _edge_proj_first(eap, BDC, btile):
    def body(ea_r, c_r, b_r, out_r):
        out_r[...] = jnp.dot(ea_r[...], c_r[...],
                             preferred_element_type=_f32) + b_r[...]
    return pl.pallas_call(
        body,
        compiler_params=pltpu.CompilerParams(skip_device_barrier=True),
        grid=(E // PK // _BM_P,),
        in_specs=[_rows((_BM_P, 128)), _full((128, 128)), _full((1, 128))],
        out_specs=_rows((_BM_P, 128)),
        out_shape=jax.ShapeDtypeStruct((E // PK, 128), _f32),
    )(eap, BDC, btile)


def _edge_proj(eap, BDW, BDC, btile):
    def body(ea_r, wm_r, c_r, b_r, t_r, cc_r):
        v = ea_r[...]
        tp = jnp.dot(v, wm_r[...], preferred_element_type=_f32)
        t_r[...] = tp.reshape(_BM_P * PK, EMB)
        cc_r[...] = jnp.dot(v, c_r[...], preferred_element_type=_f32) + b_r[...]
    return pl.pallas_call(
        body,
        compiler_params=pltpu.CompilerParams(skip_device_barrier=True),
        grid=(E // PK // _BM_P,),
        in_specs=[_rows((_BM_P, 128)), _full((128, PK * EMB)),
                  _full((128, 128)), _full((1, 128))],
        out_specs=[_rows((_BM_P * PK, EMB)), _rows((_BM_P, 128))],
        out_shape=[jax.ShapeDtypeStruct((E, EMB), _f32),
                   jax.ShapeDtypeStruct((E // PK, 128), _f32)],
    )(eap, BDW, BDC, btile)


def _edge_head_sum(zp, BDones):
    def body(z_r, w_r, out_r):
        out_r[...] = jnp.dot(z_r[...], w_r[...], preferred_element_type=_f32)
    return pl.pallas_call(
        body,
        compiler_params=pltpu.CompilerParams(skip_device_barrier=True),
        grid=(E // PK // _BM_P,),
        in_specs=[_rows((_BM_P, 128)), _full((128, PK))],
        out_specs=_rows((_BM_P, PK)),
        out_shape=jax.ShapeDtypeStruct((E // PK, PK), _f32),
    )(zp, BDones)


def _sc_params():
    return pltpu.CompilerParams(use_tc_tiling_on_sc=False,
                                skip_device_barrier=True)


def _chunk_off(base, ci):
    return pl.multiple_of(base + ci * CK, 8)


@functools.partial(
    pl.kernel,
    out_type=jax.ShapeDtypeStruct((E // PK, 128), _f32),
    mesh=_mesh,
    compiler_params=_sc_params(),
    scratch_types=(
        [pltpu.VMEM((NCHUNK, CK), jnp.int32)] * 2
        + [pltpu.VMEM((CK, ED), _f32)] * (2 * NBUF)
        + [pltpu.VMEM((CKR, 128), _f32)] * (2 * NBUF)
        + [pltpu.SemaphoreType.DMA] * (4 * NBUF)
    ),
)
def _sc_edge_update(p_hbm, q_hbm, c_hbm, src3_hbm, dst3_hbm, out_hbm, *sc):
    srcb, dstb = sc[0], sc[1]
    PS = sc[2:2 + NBUF]
    QD = sc[2 + NBUF:2 + 2 * NBUF]
    CB = sc[2 + 2 * NBUF:2 + 3 * NBUF]
    OB = sc[2 + 3 * NBUF:2 + 4 * NBUF]
    base_s = 2 + 4 * NBUF
    SP = sc[base_s:base_s + NBUF]
    SQ = sc[base_s + NBUF:base_s + 2 * NBUF]
    SCM = sc[base_s + 2 * NBUF:base_s + 3 * NBUF]
    SW = sc[base_s + 3 * NBUF:base_s + 4 * NBUF]

    wid = lax.axis_index("c") * NS + lax.axis_index("s")
    base = wid * EPW
    rbase = wid * RPW
    pltpu.sync_copy(src3_hbm.at[wid], srcb)
    pltpu.sync_copy(dst3_hbm.at[wid], dstb)

    def issue(ci, b):
        roff = rbase + ci * CKR
        pltpu.async_copy(p_hbm.at[srcb.at[ci]], PS[b], SP[b])
        pltpu.async_copy(q_hbm.at[dstb.at[ci]], QD[b], SQ[b])
        pltpu.async_copy(c_hbm.at[pl.ds(roff, CKR)], CB[b], SCM[b])

    def wait_wb(b, roff):
        pltpu.make_async_copy(OB[b], out_hbm.at[pl.ds(roff, CKR)], SW[b]).wait()

    for b in range(NBUF):
        issue(b, b)

    def step(ci, b, wb_cond, issue_next):
        roff = rbase + ci * CKR
        pltpu.make_async_copy(p_hbm.at[srcb.at[ci]], PS[b], SP[b]).wait()
        pltpu.make_async_copy(q_hbm.at[dstb.at[ci]], QD[b], SQ[b]).wait()
        pltpu.make_async_copy(c_hbm.at[pl.ds(roff, CKR)], CB[b], SCM[b]).wait()
        if wb_cond is True:
            wait_wb(b, roff)
        elif wb_cond is not None:
            pl.when(wb_cond)(lambda: wait_wb(b, roff))

        def ebody(k8, c2):
            for e in range(PK):
                sl = pl.ds(e * ED, ED)
                OB[b][k8, sl] = jnp.maximum(
                    PS[b][k8 * PK + e] + QD[b][k8 * PK + e] + CB[b][k8, sl],
                    0.0)
            return c2
        lax.fori_loop(0, CKR, ebody, 0)
        pltpu.async_copy(OB[b], out_hbm.at[pl.ds(roff, CKR)], SW[b])
        if issue_next:
            pl.when(ci + NBUF < NCHUNK)(lambda: issue(ci + NBUF, b))

    def outer(io, c2):
        for b in range(NBUF):
            step(NBUF * io + b, b, wb_cond=(io >= 1), issue_next=True)
        return c2

    lax.fori_loop(0, (NCHUNK - 1) // NBUF, outer, 0)
    step(NCHUNK - 1, (NCHUNK - 1) % NBUF, wb_cond=True, issue_next=False)
    for b in range(NBUF):
        wait_wb(b, rbase)


@functools.partial(
    pl.kernel,
    out_type=jax.ShapeDtypeStruct((E // PK, 128), _f32),
    mesh=_mesh,
    compiler_params=_sc_params(),
    scratch_types=(
        [pltpu.VMEM((NCHUNK, CK), jnp.int32)] * 2
        + [pltpu.VMEM((CK, 2 * ED), _f32)] * (2 * NBUF)
        + [pltpu.VMEM((CKR, 128), _f32)] * (2 * NBUF)
        + [pltpu.VMEM((16,), _f32)]
        + [pltpu.SemaphoreType.DMA] * (4 * NBUF)
    ),
)
def _sc_edge_pred(s_hbm, d_hbm, c_hbm, u3_hbm, src3_hbm, dst3_hbm, out_hbm,
                  *sc):
    srcb, dstb = sc[0], sc[1]
    PS = sc[2:2 + NBUF]
    QD = sc[2 + NBUF:2 + 2 * NBUF]
    CB = sc[2 + 2 * NBUF:2 + 3 * NBUF]
    OB = sc[2 + 3 * NBUF:2 + 4 * NBUF]
    u3v = sc[2 + 4 * NBUF]
    base_s = 3 + 4 * NBUF
    SP = sc[base_s:base_s + NBUF]
    SQ = sc[base_s + NBUF:base_s + 2 * NBUF]
    SCM = sc[base_s + 2 * NBUF:base_s + 3 * NBUF]
    SW = sc[base_s + 3 * NBUF:base_s + 4 * NBUF]

    wid = lax.axis_index("c") * NS + lax.axis_index("s")
    rbase = wid * RPW
    pltpu.sync_copy(src3_hbm.at[wid], srcb)
    pltpu.sync_copy(dst3_hbm.at[wid], dstb)
    pltpu.sync_copy(u3_hbm, u3v)

    def issue(ci, b):
        roff = rbase + ci * CKR
        pltpu.async_copy(s_hbm.at[srcb.at[ci]], PS[b], SP[b])
        pltpu.async_copy(d_hbm.at[dstb.at[ci]], QD[b], SQ[b])
        pltpu.async_copy(c_hbm.at[pl.ds(roff, CKR)], CB[b], SCM[b])

    def wait_wb(b, roff):
        pltpu.make_async_copy(OB[b], out_hbm.at[pl.ds(roff, CKR)], SW[b]).wait()

    for b in range(NBUF):
        issue(b, b)

    def step(ci, b, wb_cond, issue_next):
        roff = rbase + ci * CKR
        pltpu.make_async_copy(s_hbm.at[srcb.at[ci]], PS[b], SP[b]).wait()
        pltpu.make_async_copy(d_hbm.at[dstb.at[ci]], QD[b], SQ[b]).wait()
        pltpu.make_async_copy(c_hbm.at[pl.ds(roff, CKR)], CB[b], SCM[b]).wait()
        if wb_cond is True:
            wait_wb(b, roff)
        elif wb_cond is not None:
            pl.when(wb_cond)(lambda: wait_wb(b, roff))

        def ebody(k8, c2):
            for e in range(PK):
                k = k8 * PK + e
                sl = pl.ds(e * ED, ED)
                pa = PS[b][k, pl.ds(0, ED)]
                pb = PS[b][k, pl.ds(ED, ED)]
                qa = QD[b][k, pl.ds(0, ED)]
                qb = QD[b][k, pl.ds(ED, ED)]
                ea2 = jnp.maximum(pa + qa + CB[b][k8, sl], 0.0)
                OB[b][k8, sl] = ea2 * u3v[...] + pb + qb
            return c2
        lax.fori_loop(0, CKR, ebody, 0)
        pltpu.async_copy(OB[b], out_hbm.at[pl.ds(roff, CKR)], SW[b])
        if issue_next:
            pl.when(ci + NBUF < NCHUNK)(lambda: issue(ci + NBUF, b))

    def outer(io, c2):
        for b in range(NBUF):
            step(NBUF * io + b, b, wb_cond=(io >= 1), issue_next=True)
        return c2

    lax.fori_loop(0, (NCHUNK - 1) // NBUF, outer, 0)
    step(NCHUNK - 1, (NCHUNK - 1) % NBUF, wb_cond=True, issue_next=False)
    for b in range(NBUF):
        wait_wb(b, rbase)


@functools.partial(
    pl.kernel,
    out_type=jax.ShapeDtypeStruct((NC, N, EMB), _f32),
    mesh=_mesh,
    compiler_params=_sc_params(),
    scratch_types=(
        [pltpu.VMEM_SHARED((N, EMB), _f32)]
        + [pltpu.VMEM((ZROWS, EMB), _f32)]
        + [pltpu.VMEM((NCHM, CKM), jnp.int32)] * 2
        + [pltpu.VMEM((CKM, EMB), _f32)] * (2 * NBUFM)
        + [pltpu.SemaphoreType.DMA] * (2 * NBUFM)
    ),
)
def _sc_msg_agg(g_hbm, t_hbm, src3_hbm, dst3_hbm, out_hbm, *sc):
    agg_sh, zb = sc[0], sc[1]
    srcb, dstb = sc[2], sc[3]
    ROWS = sc[4:4 + NBUFM]
    TB = sc[4 + NBUFM:4 + 2 * NBUFM]
    SG = sc[4 + 2 * NBUFM:4 + 3 * NBUFM]
    ST = sc[4 + 3 * NBUFM:4 + 4 * NBUFM]

    cid = lax.axis_index("c")
    sid = lax.axis_index("s")
    wid = cid * NS + sid
    rbase = wid * RPW
    pltpu.sync_copy(src3_hbm.at[wid], srcb)
    pltpu.sync_copy(dst3_hbm.at[wid], dstb)

    def zrow(k, carry):
        for j in range(EMB // 16):
            zb[k, pl.ds(j * 16, 16)] = jnp.zeros((16,), _f32)
        return carry
    lax.fori_loop(0, ZROWS, zrow, 0)

    def zcp(i, carry):
        pltpu.sync_copy(zb, agg_sh.at[pl.ds(sid * RPT + i * ZROWS, ZROWS)])
        return carry
    lax.fori_loop(0, RPT // ZROWS, zcp, 0)
    plsc.subcore_barrier()

    base = wid * EPW

    def issue(ci, b):
        off = base + ci * CKM
        pltpu.async_copy(g_hbm.at[dstb.at[ci]], ROWS[b], SG[b])
        pltpu.async_copy(t_hbm.at[pl.ds(pl.multiple_of(off, 8), CKM)],
                         TB[b], ST[b])

    for b in range(NBUFM):
        issue(b, b)

    def step(ci, b):
        off = base + ci * CKM
        pltpu.make_async_copy(g_hbm.at[dstb.at[ci]], ROWS[b], SG[b]).wait()
        pltpu.make_async_copy(t_hbm.at[pl.ds(pl.multiple_of(off, 8), CKM)],
                              TB[b], ST[b]).wait()

        def ebody(k, c2):
            for j in range(EMB // 16):
                sl = pl.ds(j * 16, 16)
                ROWS[b][k, sl] = jnp.maximum(ROWS[b][k, sl] + TB[b][k, sl], 0.0)
            return c2
        lax.fori_loop(0, CKM, ebody, 0)
        pltpu.sync_copy(ROWS[b], agg_sh.at[srcb.at[ci]], add=True)
        pl.when(ci + NBUFM < NCHM)(lambda: issue(ci + NBUFM, b))

    def outer(io, c2):
        for b in range(NBUFM):
            step(NBUFM * io + b, b)
        return c2

    lax.fori_loop(0, NCHM // NBUFM, outer, 0)
    plsc.subcore_barrier()
    pltpu.sync_copy(agg_sh.at[pl.ds(sid * RPT, RPT)],
                    out_hbm.at[cid, pl.ds(sid * RPT, RPT)])


def kernel(x, edge_attr, edge_index, W_emb, b_emb, W_msg, b_msg, W_upd, b_upd,
           W_eu, b_eu, W_np, b_np, W_ep, b_ep):
    src = edge_index[0]
    dst = edge_index[1]
    src3 = src.reshape(NW, NCHUNK, CK)
    dst3 = dst.reshape(NW, NCHUNK, CK)
    srcm = src.reshape(NW, NCHM, CKM)
    dstm = dst.reshape(NW, NCHM, CKM)

    def r2(b):
        return b.reshape(1, -1)

    A = [W_eu[l][:EMB] for l in range(3)]
    B = [W_eu[l][EMB:2 * EMB] for l in range(3)]
    C = [W_eu[l][2 * EMB:] for l in range(3)]
    Wm1 = [W_msg[l][:EMB] for l in range(2)]
    Wm2 = [W_msg[l][EMB:] for l in range(2)]
    Wu1 = [W_upd[l][:EMB] for l in range(2)]
    Wu2 = [W_upd[l][EMB:] for l in range(2)]

    zpad = jnp.zeros((EMB, ED - 1), _f32)
    WS = jnp.concatenate([A[2], W_ep[:EMB], zpad], axis=1)
    WD = jnp.concatenate([B[2], W_ep[EMB:2 * EMB], zpad], axis=1)
    bD = jnp.zeros((2 * ED,), _f32).at[ED + 1].set(b_ep[0])
    u3 = W_ep[2 * EMB:, 0]

    eye8 = jnp.eye(PK, dtype=_f32)
    bdC = [jnp.kron(eye8, C[l]) for l in range(3)]
    bdW = [jnp.kron(eye8, Wm2[l]) for l in range(2)]
    btile = [r2(jnp.tile(b_eu[l], PK)) for l in range(3)]
    bdones = jnp.kron(eye8, jnp.ones((ED, 1), _f32))
    eap = edge_attr.reshape(E // PK, PK * ED)

    h0, P0, Q0, g0 = _node_emb_proj(x, W_emb, r2(b_emb), A[0], B[0],
                                    Wm1[0], r2(b_msg[0]))
    c0 = _edge_proj_first(eap, bdC[0], btile[0])
    ea0 = _sc_edge_update(P0, Q0, c0, src3, dst3)
    t0, c1 = _edge_proj(ea0, bdW[0], bdC[1], btile[1])
    pagg0 = _sc_msg_agg(g0, t0, srcm, dstm)

    h1, P1, Q1, g1 = _node_update_proj(h0, pagg0, Wu1[0], Wu2[0],
                                       r2(b_upd[0]), A[1], B[1],
                                       Wm1[1], r2(b_msg[1]))
    ea1 = _sc_edge_update(P1, Q1, c1, src3, dst3)
    t1, c2 = _edge_proj(ea1, bdW[1], bdC[2], btile[2])
    pagg1 = _sc_msg_agg(g1, t1, srcm, dstm)

    npred, S2, D2 = _node_final(h1, pagg1, Wu1[1], Wu2[1], r2(b_upd[1]),
                                W_np, r2(b_np), WS, WD, r2(bD))
    z16 = _sc_edge_pred(S2, D2, c2, u3, src3, dst3)
    return (npred, _edge_head_sum(z16, bdones).reshape(E, 1))


# speedup vs baseline: 6.1378x; 1.0469x over previous
import functools

import jax
import jax.numpy as jnp
from jax import lax
from jax.experimental import pallas as pl
from jax.experimental.pallas import tpu as pltpu
from jax.experimental.pallas import tpu_sc as plsc

N = 10000
E = 320000
EMB = 128
ED = 16
OUT_DIM = 128

NC = 2
NS = 16
NW = NC * NS
EPW = E // NW
CK = 80
NCHUNK = EPW // CK
NBUF = 4
CKM = 40
NCHM = EPW // CKM
NBUFM = 2
RPT = N // NS
ZROWS = 25

PK = 8
RPW = EPW // PK
CKR = CK // PK
CKMR = CKM // PK

_mesh = plsc.VectorSubcoreMesh(
    core_axis_name="c", subcore_axis_name="s", num_cores=NC, num_subcores=NS)

_f32 = jnp.float32


_BM_N = 2000
_BM_E = 2000


def _full(shape):
    return pl.BlockSpec(shape, lambda i: (0,) * len(shape))


def _rows(shape):
    return pl.BlockSpec(shape, lambda i: (i,) + (0,) * (len(shape) - 1))


def _node_emb_proj(x, We, be, A, B, Wm1, bm):
    def body(x_r, we_r, be_r, a_r, b_r, wm_r, bm_r, h_r, p_r, q_r, g_r):
        h = jnp.dot(x_r[...], we_r[...], preferred_element_type=_f32) + be_r[...]
        h_r[...] = h
        p_r[...] = jnp.dot(h, a_r[...], preferred_element_type=_f32)
        q_r[...] = jnp.dot(h, b_r[...], preferred_element_type=_f32)
        g_r[...] = jnp.dot(h, wm_r[...], preferred_element_type=_f32) + bm_r[...]
    return pl.pallas_call(
        body,
        compiler_params=pltpu.CompilerParams(skip_device_barrier=True),
        grid=(N // _BM_N,),
        in_specs=[_rows((_BM_N, EMB)), _full((EMB, EMB)), _full((1, EMB)),
                  _full((EMB, ED)), _full((EMB, ED)), _full((EMB, EMB)),
                  _full((1, EMB))],
        out_specs=[_rows((_BM_N, EMB)), _rows((_BM_N, ED)), _rows((_BM_N, ED)),
                   _rows((_BM_N, EMB))],
        out_shape=[jax.ShapeDtypeStruct((N, EMB), _f32),
                   jax.ShapeDtypeStruct((N, ED), _f32),
                   jax.ShapeDtypeStruct((N, ED), _f32),
                   jax.ShapeDtypeStruct((N, EMB), _f32)],
    )(x, We, be, A, B, Wm1, bm)


def _node_update_proj(h, pagg, Wu1, Wu2, bu, A, B, Wm1, bm):
    def body(h_r, pa_r, wu1_r, wu2_r, bu_r, a_r, b_r, wm_r, bm_r,
             hn_r, p_r, q_r, g_r):
        agg = pa_r[0] + pa_r[1]
        hn = jnp.maximum(
            jnp.dot(h_r[...], wu1_r[...], preferred_element_type=_f32)
            + jnp.dot(agg, wu2_r[...], preferred_element_type=_f32)
            + bu_r[...], 0.0)
        hn_r[...] = hn
        p_r[...] = jnp.dot(hn, a_r[...], preferred_element_type=_f32)
        q_r[...] = jnp.dot(hn, b_r[...], preferred_element_type=_f32)
        g_r[...] = jnp.dot(hn, wm_r[...], preferred_element_type=_f32) + bm_r[...]
    return pl.pallas_call(
        body,
        compiler_params=pltpu.CompilerParams(skip_device_barrier=True),
        grid=(N // _BM_N,),
        in_specs=[_rows((_BM_N, EMB)),
                  pl.BlockSpec((NC, _BM_N, EMB), lambda i: (0, i, 0)),
                  _full((EMB, EMB)), _full((EMB, EMB)), _full((1, EMB)),
                  _full((EMB, ED)), _full((EMB, ED)), _full((EMB, EMB)),
                  _full((1, EMB))],
        out_specs=[_rows((_BM_N, EMB)), _rows((_BM_N, ED)), _rows((_BM_N, ED)),
                   _rows((_BM_N, EMB))],
        out_shape=[jax.ShapeDtypeStruct((N, EMB), _f32),
                   jax.ShapeDtypeStruct((N, ED), _f32),
                   jax.ShapeDtypeStruct((N, ED), _f32),
                   jax.ShapeDtypeStruct((N, EMB), _f32)],
    )(h, pagg, Wu1, Wu2, bu, A, B, Wm1, bm)


def _node_final(h, pagg, Wu1, Wu2, bu, Wnp, bnp, WS, WD, bD):
    def body(h_r, pa_r, wu1_r, wu2_r, bu_r, wnp_r, bnp_r, ws_r, wd_r, bd_r,
             np_r, s_r, d_r):
        agg = pa_r[0] + pa_r[1]
        hn = jnp.maximum(
            jnp.dot(h_r[...], wu1_r[...], preferred_element_type=_f32)
            + jnp.dot(agg, wu2_r[...], preferred_element_type=_f32)
            + bu_r[...], 0.0)
        np_r[...] = jnp.dot(hn, wnp_r[...], preferred_element_type=_f32) + bnp_r[...]
        s_r[...] = jnp.dot(hn, ws_r[...], preferred_element_type=_f32)
        d_r[...] = jnp.dot(hn, wd_r[...], preferred_element_type=_f32) + bd_r[...]
    return pl.pallas_call(
        body,
        compiler_params=pltpu.CompilerParams(skip_device_barrier=True),
        grid=(N // _BM_N,),
        in_specs=[_rows((_BM_N, EMB)),
                  pl.BlockSpec((NC, _BM_N, EMB), lambda i: (0, i, 0)),
                  _full((EMB, EMB)), _full((EMB, EMB)), _full((1, EMB)),
                  _full((EMB, OUT_DIM)), _full((1, OUT_DIM)),
                  _full((EMB, 2 * ED)), _full((EMB, 2 * ED)), _full((1, 2 * ED))],
        out_specs=[_rows((_BM_N, OUT_DIM)), _rows((_BM_N, 2 * ED)),
                   _rows((_BM_N, 2 * ED))],
        out_shape=[jax.ShapeDtypeStruct((N, OUT_DIM), _f32),
                   jax.ShapeDtypeStruct((N, 2 * ED), _f32),
                   jax.ShapeDtypeStruct((N, 2 * ED), _f32)],
    )(h, pagg, Wu1, Wu2, bu, Wnp, bnp, WS, WD, bD)


_BM_P = 2000


def _edge_proj_first(eap, BDC, btile):
    def body(ea_r, c_r, b_r, out_r):
        out_r[...] = jnp.dot(ea_r[...], c_r[...],
                             preferred_element_type=_f32) + b_r[...]
    return pl.pallas_call(
        body,
        compiler_params=pltpu.CompilerParams(skip_device_barrier=True),
        grid=(E // PK // _BM_P,),
        in_specs=[_rows((_BM_P, 128)), _full((128, 128)), _full((1, 128))],
        out_specs=_rows((_BM_P, 128)),
        out_shape=jax.ShapeDtypeStruct((E // PK, 128), _f32),
    )(eap, BDC, btile)


def _edge_proj(eap, BDW, BDC, btile):
    def body(ea_r, wm_r, c_r, b_r, t_r, cc_r):
        v = ea_r[...]
        tp = jnp.dot(v, wm_r[...], preferred_element_type=_f32)
        t_r[...] = tp.reshape(_BM_P * PK, EMB)
        cc_r[...] = jnp.dot(v, c_r[...], preferred_element_type=_f32) + b_r[...]
    return pl.pallas_call(
        body,
        compiler_params=pltpu.CompilerParams(skip_device_barrier=True),
        grid=(E // PK // _BM_P,),
        in_specs=[_rows((_BM_P, 128)), _full((128, PK * EMB)),
                  _full((128, 128)), _full((1, 128))],
        out_specs=[_rows((_BM_P * PK, EMB)), _rows((_BM_P, 128))],
        out_shape=[jax.ShapeDtypeStruct((E, EMB), _f32),
                   jax.ShapeDtypeStruct((E // PK, 128), _f32)],
    )(eap, BDW, BDC, btile)


def _edge_head_sum(zp, BDones):
    def body(z_r, w_r, out_r):
        out_r[...] = jnp.dot(z_r[...], w_r[...], preferred_element_type=_f32)
    return pl.pallas_call(
        body,
        compiler_params=pltpu.CompilerParams(skip_device_barrier=True),
        grid=(E // PK // _BM_P,),
        in_specs=[_rows((_BM_P, 128)), _full((128, PK))],
        out_specs=_rows((_BM_P, PK)),
        out_shape=jax.ShapeDtypeStruct((E // PK, PK), _f32),
    )(zp, BDones)


def _sc_params():
    return pltpu.CompilerParams(use_tc_tiling_on_sc=False,
                                skip_device_barrier=True)


def _chunk_off(base, ci):
    return pl.multiple_of(base + ci * CK, 8)


@functools.partial(
    pl.kernel,
    out_type=jax.ShapeDtypeStruct((E // PK, 128), _f32),
    mesh=_mesh,
    compiler_params=_sc_params(),
    scratch_types=(
        [pltpu.VMEM((NCHUNK, CK), jnp.int32)] * 2
        + [pltpu.VMEM((CK, ED), _f32)] * (2 * NBUF)
        + [pltpu.VMEM((CKR, 128), _f32)] * (2 * NBUF)
        + [pltpu.SemaphoreType.DMA] * (4 * NBUF)
    ),
)
def _sc_edge_update(p_hbm, q_hbm, c_hbm, src3_hbm, dst3_hbm, out_hbm, *sc):
    srcb, dstb = sc[0], sc[1]
    PS = sc[2:2 + NBUF]
    QD = sc[2 + NBUF:2 + 2 * NBUF]
    CB = sc[2 + 2 * NBUF:2 + 3 * NBUF]
    OB = sc[2 + 3 * NBUF:2 + 4 * NBUF]
    base_s = 2 + 4 * NBUF
    SP = sc[base_s:base_s + NBUF]
    SQ = sc[base_s + NBUF:base_s + 2 * NBUF]
    SCM = sc[base_s + 2 * NBUF:base_s + 3 * NBUF]
    SW = sc[base_s + 3 * NBUF:base_s + 4 * NBUF]

    wid = lax.axis_index("c") * NS + lax.axis_index("s")
    base = wid * EPW
    rbase = wid * RPW
    pltpu.sync_copy(src3_hbm.at[wid], srcb)
    pltpu.sync_copy(dst3_hbm.at[wid], dstb)

    def issue(ci, b):
        roff = rbase + ci * CKR
        pltpu.async_copy(p_hbm.at[srcb.at[ci]], PS[b], SP[b])
        pltpu.async_copy(q_hbm.at[dstb.at[ci]], QD[b], SQ[b])
        pltpu.async_copy(c_hbm.at[pl.ds(roff, CKR)], CB[b], SCM[b])

    def wait_wb(b, roff):
        pltpu.make_async_copy(OB[b], out_hbm.at[pl.ds(roff, CKR)], SW[b]).wait()

    for b in range(NBUF):
        issue(b, b)

    def step(ci, b, wb_cond, issue_next):
        roff = rbase + ci * CKR
        pltpu.make_async_copy(p_hbm.at[srcb.at[ci]], PS[b], SP[b]).wait()
        pltpu.make_async_copy(q_hbm.at[dstb.at[ci]], QD[b], SQ[b]).wait()
        pltpu.make_async_copy(c_hbm.at[pl.ds(roff, CKR)], CB[b], SCM[b]).wait()
        if wb_cond is True:
            wait_wb(b, roff)
        elif wb_cond is not None:
            pl.when(wb_cond)(lambda: wait_wb(b, roff))

        def ebody(k8, c2):
            for e in range(PK):
                sl = pl.ds(e * ED, ED)
                OB[b][k8, sl] = jnp.maximum(
                    PS[b][k8 * PK + e] + QD[b][k8 * PK + e] + CB[b][k8, sl],
                    0.0)
            return c2
        lax.fori_loop(0, CKR, ebody, 0)
        pltpu.async_copy(OB[b], out_hbm.at[pl.ds(roff, CKR)], SW[b])
        if issue_next:
            pl.when(ci + NBUF < NCHUNK)(lambda: issue(ci + NBUF, b))

    def outer(io, c2):
        for b in range(NBUF):
            step(NBUF * io + b, b, wb_cond=(io >= 1), issue_next=True)
        return c2

    lax.fori_loop(0, (NCHUNK - 1) // NBUF, outer, 0)
    step(NCHUNK - 1, (NCHUNK - 1) % NBUF, wb_cond=True, issue_next=False)
    for b in range(NBUF):
        wait_wb(b, rbase)


@functools.partial(
    pl.kernel,
    out_type=jax.ShapeDtypeStruct((E // PK, 128), _f32),
    mesh=_mesh,
    compiler_params=_sc_params(),
    scratch_types=(
        [pltpu.VMEM((NCHUNK, CK), jnp.int32)] * 2
        + [pltpu.VMEM((CK, 2 * ED), _f32)] * (2 * NBUF)
        + [pltpu.VMEM((CKR, 128), _f32)] * (2 * NBUF)
        + [pltpu.VMEM((16,), _f32)]
        + [pltpu.SemaphoreType.DMA] * (4 * NBUF)
    ),
)
def _sc_edge_pred(s_hbm, d_hbm, c_hbm, u3_hbm, src3_hbm, dst3_hbm, out_hbm,
                  *sc):
    srcb, dstb = sc[0], sc[1]
    PS = sc[2:2 + NBUF]
    QD = sc[2 + NBUF:2 + 2 * NBUF]
    CB = sc[2 + 2 * NBUF:2 + 3 * NBUF]
    OB = sc[2 + 3 * NBUF:2 + 4 * NBUF]
    u3v = sc[2 + 4 * NBUF]
    base_s = 3 + 4 * NBUF
    SP = sc[base_s:base_s + NBUF]
    SQ = sc[base_s + NBUF:base_s + 2 * NBUF]
    SCM = sc[base_s + 2 * NBUF:base_s + 3 * NBUF]
    SW = sc[base_s + 3 * NBUF:base_s + 4 * NBUF]

    wid = lax.axis_index("c") * NS + lax.axis_index("s")
    rbase = wid * RPW
    pltpu.sync_copy(src3_hbm.at[wid], srcb)
    pltpu.sync_copy(dst3_hbm.at[wid], dstb)
    pltpu.sync_copy(u3_hbm, u3v)

    def issue(ci, b):
        roff = rbase + ci * CKR
        pltpu.async_copy(s_hbm.at[srcb.at[ci]], PS[b], SP[b])
        pltpu.async_copy(d_hbm.at[dstb.at[ci]], QD[b], SQ[b])
        pltpu.async_copy(c_hbm.at[pl.ds(roff, CKR)], CB[b], SCM[b])

    def wait_wb(b, roff):
        pltpu.make_async_copy(OB[b], out_hbm.at[pl.ds(roff, CKR)], SW[b]).wait()

    for b in range(NBUF):
        issue(b, b)

    def step(ci, b, wb_cond, issue_next):
        roff = rbase + ci * CKR
        pltpu.make_async_copy(s_hbm.at[srcb.at[ci]], PS[b], SP[b]).wait()
        pltpu.make_async_copy(d_hbm.at[dstb.at[ci]], QD[b], SQ[b]).wait()
        pltpu.make_async_copy(c_hbm.at[pl.ds(roff, CKR)], CB[b], SCM[b]).wait()
        if wb_cond is True:
            wait_wb(b, roff)
        elif wb_cond is not None:
            pl.when(wb_cond)(lambda: wait_wb(b, roff))

        def ebody(k8, c2):
            for e in range(PK):
                k = k8 * PK + e
                sl = pl.ds(e * ED, ED)
                pa = PS[b][k, pl.ds(0, ED)]
                pb = PS[b][k, pl.ds(ED, ED)]
                qa = QD[b][k, pl.ds(0, ED)]
                qb = QD[b][k, pl.ds(ED, ED)]
                ea2 = jnp.maximum(pa + qa + CB[b][k8, sl], 0.0)
                OB[b][k8, sl] = ea2 * u3v[...] + pb + qb
            return c2
        lax.fori_loop(0, CKR, ebody, 0)
        pltpu.async_copy(OB[b], out_hbm.at[pl.ds(roff, CKR)], SW[b])
        if issue_next:
            pl.when(ci + NBUF < NCHUNK)(lambda: issue(ci + NBUF, b))

    def outer(io, c2):
        for b in range(NBUF):
            step(NBUF * io + b, b, wb_cond=(io >= 1), issue_next=True)
        return c2

    lax.fori_loop(0, (NCHUNK - 1) // NBUF, outer, 0)
    step(NCHUNK - 1, (NCHUNK - 1) % NBUF, wb_cond=True, issue_next=False)
    for b in range(NBUF):
        wait_wb(b, rbase)


@functools.partial(
    pl.kernel,
    out_type=jax.ShapeDtypeStruct((NC, N, EMB), _f32),
    mesh=_mesh,
    compiler_params=_sc_params(),
    scratch_types=(
        [pltpu.VMEM_SHARED((N, EMB), _f32)]
        + [pltpu.VMEM((NCHM, CKM), jnp.int32)] * 2
        + [pltpu.VMEM((CKM, EMB), _f32)] * (3 * NBUFM)
        + [pltpu.SemaphoreType.DMA] * (3 * NBUFM)
    ),
)
def _sc_msg_agg(g_hbm, t_hbm, src3_hbm, dst3_hbm, out_hbm, *sc):
    agg_sh = sc[0]
    srcb, dstb = sc[1], sc[2]
    ROWS = sc[3:3 + NBUFM]
    TB = sc[3 + NBUFM:3 + 2 * NBUFM]
    MB = sc[3 + 2 * NBUFM:3 + 3 * NBUFM]
    base_s = 3 + 3 * NBUFM
    SG = sc[base_s:base_s + NBUFM]
    ST = sc[base_s + NBUFM:base_s + 2 * NBUFM]
    SA = sc[base_s + 2 * NBUFM:base_s + 3 * NBUFM]

    cid = lax.axis_index("c")
    sid = lax.axis_index("s")
    wid = cid * NS + sid
    base = wid * EPW
    pltpu.sync_copy(src3_hbm.at[wid], srcb)
    pltpu.sync_copy(dst3_hbm.at[wid], dstb)

    def zrow(k, carry):
        for j in range(EMB // 16):
            MB[0][k, pl.ds(j * 16, 16)] = jnp.zeros((16,), _f32)
        return carry
    lax.fori_loop(0, CKM, zrow, 0)

    def zcp(i, carry):
        pltpu.sync_copy(MB[0], agg_sh.at[pl.ds(sid * RPT + i * CKM, CKM)])
        return carry
    lax.fori_loop(0, RPT // CKM, zcp, 0)
    pltpu.sync_copy(MB[0].at[pl.ds(0, RPT % CKM)],
                    agg_sh.at[pl.ds(sid * RPT + (RPT // CKM) * CKM,
                                    RPT % CKM)])
    plsc.subcore_barrier()

    def issue(ci, b):
        off = base + ci * CKM
        pltpu.async_copy(g_hbm.at[dstb.at[ci]], ROWS[b], SG[b])
        pltpu.async_copy(t_hbm.at[pl.ds(pl.multiple_of(off, 8), CKM)],
                         TB[b], ST[b])

    for b in range(NBUFM):
        issue(b, b)

    def drain_sa(ci, b):
        pltpu.make_async_copy(MB[b], agg_sh.at[srcb.at[ci]], SA[b]).wait()

    def step(ci, b, sa_cond):
        off = base + ci * CKM
        pltpu.make_async_copy(g_hbm.at[dstb.at[ci]], ROWS[b], SG[b]).wait()
        pltpu.make_async_copy(t_hbm.at[pl.ds(pl.multiple_of(off, 8), CKM)],
                              TB[b], ST[b]).wait()
        if sa_cond is True:
            drain_sa(ci, b)
        elif sa_cond is not None:
            pl.when(sa_cond)(lambda: drain_sa(ci, b))

        def ebody(k, c2):
            for j in range(EMB // 16):
                sl = pl.ds(j * 16, 16)
                MB[b][k, sl] = jnp.maximum(ROWS[b][k, sl] + TB[b][k, sl], 0.0)
            return c2
        lax.fori_loop(0, CKM, ebody, 0)
        pltpu.make_async_copy(MB[b], agg_sh.at[srcb.at[ci]],
                              SA[b]).start(add=True)
        pl.when(ci + NBUFM < NCHM)(lambda: issue(ci + NBUFM, b))

    def outer(io, c2):
        for b in range(NBUFM):
            step(NBUFM * io + b, b, sa_cond=(io >= 1))
        return c2

    lax.fori_loop(0, NCHM // NBUFM, outer, 0)
    for b in range(NBUFM):
        drain_sa(0, b)
    plsc.subcore_barrier()
    pltpu.sync_copy(agg_sh.at[pl.ds(sid * RPT, RPT)],
                    out_hbm.at[cid, pl.ds(sid * RPT, RPT)])


def kernel(x, edge_attr, edge_index, W_emb, b_emb, W_msg, b_msg, W_upd, b_upd,
           W_eu, b_eu, W_np, b_np, W_ep, b_ep):
    src = edge_index[0]
    dst = edge_index[1]
    src3 = src.reshape(NW, NCHUNK, CK)
    dst3 = dst.reshape(NW, NCHUNK, CK)
    srcm = src.reshape(NW, NCHM, CKM)
    dstm = dst.reshape(NW, NCHM, CKM)

    def r2(b):
        return b.reshape(1, -1)

    A = [W_eu[l][:EMB] for l in range(3)]
    B = [W_eu[l][EMB:2 * EMB] for l in range(3)]
    C = [W_eu[l][2 * EMB:] for l in range(3)]
    Wm1 = [W_msg[l][:EMB] for l in range(2)]
    Wm2 = [W_msg[l][EMB:] for l in range(2)]
    Wu1 = [W_upd[l][:EMB] for l in range(2)]
    Wu2 = [W_upd[l][EMB:] for l in range(2)]

    zpad = jnp.zeros((EMB, ED - 1), _f32)
    WS = jnp.concatenate([A[2], W_ep[:EMB], zpad], axis=1)
    WD = jnp.concatenate([B[2], W_ep[EMB:2 * EMB], zpad], axis=1)
    bD = jnp.zeros((2 * ED,), _f32).at[ED + 1].set(b_ep[0])
    u3 = W_ep[2 * EMB:, 0]

    eye8 = jnp.eye(PK, dtype=_f32)
    bdC = [jnp.kron(eye8, C[l]) for l in range(3)]
    bdW = [jnp.kron(eye8, Wm2[l]) for l in range(2)]
    btile = [r2(jnp.tile(b_eu[l], PK)) for l in range(3)]
    bdones = jnp.kron(eye8, jnp.ones((ED, 1), _f32))
    eap = edge_attr.reshape(E // PK, PK * ED)

    h0, P0, Q0, g0 = _node_emb_proj(x, W_emb, r2(b_emb), A[0], B[0],
                                    Wm1[0], r2(b_msg[0]))
    c0 = _edge_proj_first(eap, bdC[0], btile[0])
    ea0 = _sc_edge_update(P0, Q0, c0, src3, dst3)
    t0, c1 = _edge_proj(ea0, bdW[0], bdC[1], btile[1])
    pagg0 = _sc_msg_agg(g0, t0, srcm, dstm)

    h1, P1, Q1, g1 = _node_update_proj(h0, pagg0, Wu1[0], Wu2[0],
                                       r2(b_upd[0]), A[1], B[1],
                                       Wm1[1], r2(b_msg[1]))
    ea1 = _sc_edge_update(P1, Q1, c1, src3, dst3)
    t1, c2 = _edge_proj(ea1, bdW[1], bdC[2], btile[2])
    pagg1 = _sc_msg_agg(g1, t1, srcm, dstm)

    npred, S2, D2 = _node_final(h1, pagg1, Wu1[1], Wu2[1], r2(b_upd[1]),
                                W_np, r2(b_np), WS, WD, r2(bD))
    z16 = _sc_edge_pred(S2, D2, c2, u3, src3, dst3)
    return (npred, _edge_head_sum(z16, bdones).reshape(E, 1))
